# baseline (speedup 1.0000x reference)
import functools

import jax
import jax.numpy as jnp
from jax.experimental import pallas as pl
from jax.experimental.pallas import tpu as pltpu


def _dot(a, b):
    return jax.lax.dot_general(a, b, (((1,), (0,)), ((), ())),
                               preferred_element_type=jnp.float32)


def _edge_body(xr_ref, xc_ref, ew1t, ew1b, eb1, ew2, eb2,
               n1w1t, n1w1b, n1b1, n1w2, n1b2, ea_ref, m_ref):
    xr = xr_ref[...]
    xc = xc_ref[...]
    h1 = jnp.maximum(_dot(xr, ew1t[...]) + _dot(xc, ew1b[...]) + eb1[...], 0.0)
    ea = _dot(h1, ew2[...]) + eb2[...]
    ea_ref[...] = ea
    h2 = jnp.maximum(_dot(xr, n1w1t[...]) + _dot(ea, n1w1b[...]) + n1b1[...], 0.0)
    m_ref[...] = _dot(h2, n1w2[...]) + n1b2[...]


def _edge_stage(xr, xc, ew1, eb1, ew2, eb2, n1w1, n1b1, n1w2, n1b2, block_e):
    e = xr.shape[0]
    grid = e // block_e
    full = lambda s: pl.BlockSpec(s, lambda i: (0, 0))
    ea, m = pl.pallas_call(
        _edge_body,
        grid=(grid,),
        in_specs=[
            pl.BlockSpec((block_e, 48), lambda i: (i, 0)),
            pl.BlockSpec((block_e, 48), lambda i: (i, 0)),
            full((48, 128)), full((48, 128)), full((1, 128)),
            full((128, 128)), full((1, 128)),
            full((48, 128)), full((128, 128)), full((1, 128)),
            full((128, 128)), full((1, 128)),
        ],
        out_specs=[
            pl.BlockSpec((block_e, 128), lambda i: (i, 0)),
            pl.BlockSpec((block_e, 128), lambda i: (i, 0)),
        ],
        out_shape=[
            jax.ShapeDtypeStruct((e, 128), jnp.float32),
            jax.ShapeDtypeStruct((e, 128), jnp.float32),
        ],
    )(xr, xc, ew1[:48], ew1[48:], eb1.reshape(1, 128), ew2, eb2.reshape(1, 128),
      n1w1[:48], n1w1[48:], n1b1.reshape(1, 128), n1w2, n1b2.reshape(1, 128))
    return ea, m


def _node_body(x_ref, agg_ref, b_ref, n2w1t, n2w1b, n2b1, n2w2, n2b2,
               gw1, gb1, gw2p, gb2p, u_ref, gsum, gcnt, *, nblocks, num_graphs):
    i = pl.program_id(0)

    @pl.when(i == 0)
    def _():
        gsum[...] = jnp.zeros_like(gsum)
        gcnt[...] = jnp.zeros_like(gcnt)

    x = x_ref[...]
    agg = agg_ref[...]
    h = jnp.maximum(_dot(x, n2w1t[...]) + _dot(agg, n2w1b[...]) + n2b1[...], 0.0)
    x2 = _dot(h, n2w2[...]) + n2b2[...]
    b = b_ref[0, 0, :]
    bn = x.shape[0]
    onehot = (b[:, None] == jax.lax.broadcasted_iota(jnp.int32, (bn, num_graphs), 1)
              ).astype(jnp.float32)
    seg = lambda v: jax.lax.dot_general(onehot, v, (((0,), (0,)), ((), ())),
                                        preferred_element_type=jnp.float32)
    gsum[...] += seg(x2)
    gcnt[...] += seg(jnp.ones_like(x2))

    @pl.when(i == nblocks - 1)
    def _():
        gmean = gsum[...] / jnp.maximum(gcnt[...], 1.0)
        hg = jnp.maximum(_dot(gmean, gw1[...]) + gb1[...], 0.0)
        u_ref[...] = _dot(hg, gw2p[...]) + gb2p[...]


def _node_stage(x, agg, batch, n2w1, n2b1, n2w2, n2b2, gw1, gb1, gw2, gb2,
                block_n, num_graphs):
    n = x.shape[0]
    grid = n // block_n
    batch3d = batch.reshape(grid, 1, block_n)
    gw2p = jnp.zeros((128, 128), jnp.float32).at[:, :2].set(gw2)
    gb2p = jnp.zeros((1, 128), jnp.float32).at[0, :2].set(gb2)
    full = lambda s: pl.BlockSpec(s, lambda i: (0,) * len(s))
    u_full = pl.pallas_call(
        functools.partial(_node_body, nblocks=grid, num_graphs=num_graphs),
        grid=(grid,),
        in_specs=[
            pl.BlockSpec((block_n, 48), lambda i: (i, 0)),
            pl.BlockSpec((block_n, 128), lambda i: (i, 0)),
            pl.BlockSpec((1, 1, block_n), lambda i: (i, 0, 0)),
            full((48, 128)), full((128, 128)), full((1, 128)),
            full((128, 128)), full((1, 128)),
            full((128, 128)), full((1, 128)), full((128, 128)), full((1, 128)),
        ],
        out_specs=pl.BlockSpec((num_graphs, 128), lambda i: (0, 0)),
        out_shape=jax.ShapeDtypeStruct((num_graphs, 128), jnp.float32),
        scratch_shapes=[
            pltpu.VMEM((num_graphs, 128), jnp.float32),
            pltpu.VMEM((num_graphs, 128), jnp.float32),
        ],
    )(x, agg, batch3d, n2w1[:48], n2w1[48:], n2b1.reshape(1, 128),
      n2w2, n2b2.reshape(1, 128), gw1, gb1.reshape(1, 128), gw2p, gb2p)
    return u_full[:, :2]


def kernel(x, edge_index, batch, ew1, eb1, ew2, eb2, n1w1, n1b1, n1w2, n1b2,
           n2w1, n2b1, n2w2, n2b2, gw1, gb1, gw2, gb2):
    n = x.shape[0]
    e = edge_index.shape[1]
    row, col = edge_index[0], edge_index[1]

    xr = jnp.take(x, row, axis=0)
    xc = jnp.take(x, col, axis=0)

    block_e = 2000 if e % 2000 == 0 else e
    ea, m = _edge_stage(xr, xc, ew1, eb1, ew2, eb2, n1w1, n1b1, n1w2, n1b2,
                        block_e)

    s = jax.ops.segment_sum(m, col, num_segments=n)
    c = jax.ops.segment_sum(jnp.ones((e, 1), jnp.float32), col, num_segments=n)
    agg = s / jnp.maximum(c, 1.0)

    block_n = 2000 if n % 2000 == 0 else n
    u = _node_stage(x, agg, batch, n2w1, n2b1, n2w2, n2b2, gw1, gb1, gw2, gb2,
                    block_n, 64)
    return (u, ea)


# speedup vs baseline: 1.0004x; 1.0004x over previous
import functools

import jax
import jax.numpy as jnp
from jax.experimental import pallas as pl
from jax.experimental.pallas import tpu as pltpu


def _dot(a, b):
    return jax.lax.dot_general(a, b, (((1,), (0,)), ((), ())),
                               preferred_element_type=jnp.float32)


def _edge_body(xr_ref, xc_ref, ew1t, ew1b, eb1, ew2, eb2,
               n1w1t, n1w1b, n1b1, n1w2, n1b2, ea_ref, m_ref):
    bf = jnp.bfloat16
    xr = xr_ref[...].astype(jnp.float32)
    xc = xc_ref[...].astype(jnp.float32)
    h1 = jnp.maximum(_dot(xr, ew1t[...]) + _dot(xc, ew1b[...]) + eb1[...], 0.0)
    ea = _dot(h1, ew2[...]) + eb2[...]
    ea_ref[...] = ea
    h2 = jnp.maximum(_dot(xr_ref[...], n1w1t[...].astype(bf))
                     + _dot(ea.astype(bf), n1w1b[...].astype(bf)) + n1b1[...], 0.0)
    m_ref[...] = _dot(h2.astype(bf), n1w2[...].astype(bf)) + n1b2[...]


def _edge_stage(xr, xc, ew1, eb1, ew2, eb2, n1w1, n1b1, n1w2, n1b2, block_e):
    e = xr.shape[0]
    grid = e // block_e
    full = lambda s: pl.BlockSpec(s, lambda i: (0, 0))
    ea, m = pl.pallas_call(
        _edge_body,
        grid=(grid,),
        in_specs=[
            pl.BlockSpec((block_e, 48), lambda i: (i, 0)),
            pl.BlockSpec((block_e, 48), lambda i: (i, 0)),
            full((48, 128)), full((48, 128)), full((1, 128)),
            full((128, 128)), full((1, 128)),
            full((48, 128)), full((128, 128)), full((1, 128)),
            full((128, 128)), full((1, 128)),
        ],
        out_specs=[
            pl.BlockSpec((block_e, 128), lambda i: (i, 0)),
            pl.BlockSpec((block_e, 128), lambda i: (i, 0)),
        ],
        out_shape=[
            jax.ShapeDtypeStruct((e, 128), jnp.float32),
            jax.ShapeDtypeStruct((e, 128), jnp.float32),
        ],
    )(xr, xc, ew1[:48], ew1[48:], eb1.reshape(1, 128), ew2, eb2.reshape(1, 128),
      n1w1[:48], n1w1[48:], n1b1.reshape(1, 128), n1w2, n1b2.reshape(1, 128))
    return ea, m


def _node_body(x_ref, agg_ref, b_ref, n2w1t, n2w1b, n2b1, n2w2, n2b2,
               gw1, gb1, gw2p, gb2p, u_ref, gsum, gcnt, *, nblocks, num_graphs):
    i = pl.program_id(0)

    @pl.when(i == 0)
    def _():
        gsum[...] = jnp.zeros_like(gsum)
        gcnt[...] = jnp.zeros_like(gcnt)

    x = x_ref[...]
    agg = agg_ref[...]
    h = jnp.maximum(_dot(x, n2w1t[...]) + _dot(agg, n2w1b[...]) + n2b1[...], 0.0)
    x2 = _dot(h, n2w2[...]) + n2b2[...]
    b = b_ref[0, 0, :]
    bn = x.shape[0]
    onehot = (b[:, None] == jax.lax.broadcasted_iota(jnp.int32, (bn, num_graphs), 1)
              ).astype(jnp.float32)
    seg = lambda v: jax.lax.dot_general(onehot, v, (((0,), (0,)), ((), ())),
                                        preferred_element_type=jnp.float32)
    gsum[...] += seg(x2)
    gcnt[...] += seg(jnp.ones_like(x2))

    @pl.when(i == nblocks - 1)
    def _():
        gmean = gsum[...] / jnp.maximum(gcnt[...], 1.0)
        hg = jnp.maximum(_dot(gmean, gw1[...]) + gb1[...], 0.0)
        u_ref[...] = _dot(hg, gw2p[...]) + gb2p[...]


def _node_stage(x, agg, batch, n2w1, n2b1, n2w2, n2b2, gw1, gb1, gw2, gb2,
                block_n, num_graphs):
    n = x.shape[0]
    grid = n // block_n
    batch3d = batch.reshape(grid, 1, block_n)
    gw2p = jnp.zeros((128, 128), jnp.float32).at[:, :2].set(gw2)
    gb2p = jnp.zeros((1, 128), jnp.float32).at[0, :2].set(gb2)
    full = lambda s: pl.BlockSpec(s, lambda i: (0,) * len(s))
    u_full = pl.pallas_call(
        functools.partial(_node_body, nblocks=grid, num_graphs=num_graphs),
        grid=(grid,),
        in_specs=[
            pl.BlockSpec((block_n, 48), lambda i: (i, 0)),
            pl.BlockSpec((block_n, 128), lambda i: (i, 0)),
            pl.BlockSpec((1, 1, block_n), lambda i: (i, 0, 0)),
            full((48, 128)), full((128, 128)), full((1, 128)),
            full((128, 128)), full((1, 128)),
            full((128, 128)), full((1, 128)), full((128, 128)), full((1, 128)),
        ],
        out_specs=pl.BlockSpec((num_graphs, 128), lambda i: (0, 0)),
        out_shape=jax.ShapeDtypeStruct((num_graphs, 128), jnp.float32),
        scratch_shapes=[
            pltpu.VMEM((num_graphs, 128), jnp.float32),
            pltpu.VMEM((num_graphs, 128), jnp.float32),
        ],
    )(x, agg, batch3d, n2w1[:48], n2w1[48:], n2b1.reshape(1, 128),
      n2w2, n2b2.reshape(1, 128), gw1, gb1.reshape(1, 128), gw2p, gb2p)
    return u_full[:, :2]


def kernel(x, edge_index, batch, ew1, eb1, ew2, eb2, n1w1, n1b1, n1w2, n1b2,
           n2w1, n2b1, n2w2, n2b2, gw1, gb1, gw2, gb2):
    n = x.shape[0]
    e = edge_index.shape[1]
    row, col = edge_index[0], edge_index[1]

    x16 = x.astype(jnp.bfloat16)
    xr = jnp.take(x16, row, axis=0)
    xc = jnp.take(x16, col, axis=0)

    block_e = 2000 if e % 2000 == 0 else e
    ea, m = _edge_stage(xr, xc, ew1, eb1, ew2, eb2, n1w1, n1b1, n1w2, n1b2,
                        block_e)

    s = jax.ops.segment_sum(m, col, num_segments=n)
    c = jax.ops.segment_sum(jnp.ones((e, 1), jnp.float32), col, num_segments=n)
    agg = s / jnp.maximum(c, 1.0)

    block_n = 2000 if n % 2000 == 0 else n
    u = _node_stage(x, agg, batch, n2w1, n2b1, n2w2, n2b2, gw1, gb1, gw2, gb2,
                    block_n, 64)
    return (u, ea)


# speedup vs baseline: 1.2327x; 1.2322x over previous
import functools

import jax
import jax.numpy as jnp
from jax import lax
from jax.experimental import pallas as pl
from jax.experimental.pallas import tpu as pltpu
from jax.experimental.pallas import tpu_sc as plsc


def _dot(a, b):
    return jax.lax.dot_general(a, b, (((1,), (0,)), ((), ())),
                               preferred_element_type=jnp.float32)


def _edge_body(xr_ref, xc_ref, ew1t, ew1b, eb1, ew2, eb2,
               n1w1t, n1w1b, n1b1, n1w2, n1b2, ea_ref, m_ref):
    bf = jnp.bfloat16
    xr = xr_ref[...].astype(jnp.float32)
    xc = xc_ref[...].astype(jnp.float32)
    h1 = jnp.maximum(_dot(xr, ew1t[...]) + _dot(xc, ew1b[...]) + eb1[...], 0.0)
    ea = _dot(h1, ew2[...]) + eb2[...]
    ea_ref[...] = ea
    h2 = jnp.maximum(_dot(xr_ref[...], n1w1t[...].astype(bf))
                     + _dot(ea.astype(bf), n1w1b[...].astype(bf)) + n1b1[...], 0.0)
    m_ref[...] = _dot(h2.astype(bf), n1w2[...].astype(bf)) + n1b2[...]


def _edge_stage(xr, xc, ew1, eb1, ew2, eb2, n1w1, n1b1, n1w2, n1b2, block_e):
    e = xr.shape[0]
    grid = e // block_e
    full = lambda s: pl.BlockSpec(s, lambda i: (0, 0))
    ea, m = pl.pallas_call(
        _edge_body,
        grid=(grid,),
        in_specs=[
            pl.BlockSpec((block_e, 48), lambda i: (i, 0)),
            pl.BlockSpec((block_e, 48), lambda i: (i, 0)),
            full((48, 128)), full((48, 128)), full((1, 128)),
            full((128, 128)), full((1, 128)),
            full((48, 128)), full((128, 128)), full((1, 128)),
            full((128, 128)), full((1, 128)),
        ],
        out_specs=[
            pl.BlockSpec((block_e, 128), lambda i: (i, 0)),
            pl.BlockSpec((block_e, 128), lambda i: (i, 0)),
        ],
        out_shape=[
            jax.ShapeDtypeStruct((e, 128), jnp.float32),
            jax.ShapeDtypeStruct((e, 128), jnp.float32),
        ],
    )(xr, xc, ew1[:48], ew1[48:], eb1.reshape(1, 128), ew2, eb2.reshape(1, 128),
      n1w1[:48], n1w1[48:], n1b1.reshape(1, 128), n1w2, n1b2.reshape(1, 128))
    return ea, m


_EC = 128


def _scatter_body(m_hbm, colr_hbm, zeros_hbm, agg_hbm, acc, colbuf, libuf, mbuf,
                  *, nch, stripe):
    c = lax.axis_index("c")
    s = lax.axis_index("s")
    nchunks = colr_hbm.shape[0]
    jmax = (nchunks + 15) // 16
    full_tiles = nch // stripe
    partial = nch - full_tiles * stripe
    for p in range(2):
        base = (2 * p + c) * nch
        pltpu.sync_copy(zeros_hbm.at[pl.ds(s * stripe, stripe)],
                        acc.at[pl.ds(s * stripe, stripe)])
        plsc.subcore_barrier()

        def eloop(j, carry):
            chunk = j * 16 + s

            @pl.when(chunk < nchunks)
            def _():
                pltpu.sync_copy(colr_hbm.at[chunk], colbuf)
                pltpu.sync_copy(m_hbm.at[pl.ds(chunk * _EC, _EC)], mbuf)
                for v in range(_EC // 16):
                    cv = colbuf[pl.ds(v * 16, 16)]
                    li = cv - base
                    memb = (li >= 0) & (li < nch)
                    libuf[pl.ds(v * 16, 16)] = jnp.where(memb, li, nch)
                pltpu.sync_copy(mbuf, acc.at[libuf], add=True)

            return carry

        lax.fori_loop(0, jmax, eloop, 0)
        plsc.subcore_barrier()
        start = s * stripe

        @pl.when(s < full_tiles)
        def _():
            pltpu.sync_copy(acc.at[pl.ds(start, stripe)],
                            agg_hbm.at[pl.ds(base + start, stripe)])

        if partial:
            @pl.when(s == full_tiles)
            def _():
                pltpu.sync_copy(acc.at[pl.ds(start, partial)],
                                agg_hbm.at[pl.ds(base + start, partial)])

        plsc.subcore_barrier()


def _scatter_stage(m, col, n):
    e = m.shape[0]
    colr = col.reshape(e // _EC, _EC)
    nch = (-(-n // 4) + 7) // 8 * 8
    stripe = (-(-(nch + 1) // 16) + 7) // 8 * 8
    accrows = stripe * 16
    zeros = jnp.zeros((accrows, 128), jnp.float32)
    mesh = plsc.VectorSubcoreMesh(core_axis_name="c", subcore_axis_name="s")
    agg = pl.kernel(
        functools.partial(_scatter_body, nch=nch, stripe=stripe),
        out_type=jax.ShapeDtypeStruct((4 * nch, 128), jnp.float32),
        mesh=mesh,
        scratch_types=[
            pltpu.VMEM_SHARED((accrows, 128), jnp.float32),
            pltpu.VMEM((_EC,), jnp.int32),
            pltpu.VMEM((_EC,), jnp.int32),
            pltpu.VMEM((_EC, 128), jnp.float32),
        ],
    )(m, colr, zeros)
    return agg


_HC = 1600


def _hist_body(colr_hbm, zeros_hbm, out_hbm, hist, colbuf):
    c = lax.axis_index("c")
    s = lax.axis_index("s")
    w = s * 2 + c
    nchunks = colr_hbm.shape[0]
    jmax = (nchunks + 31) // 32
    pltpu.sync_copy(zeros_hbm, hist)
    ones = jnp.ones((16,), jnp.float32)

    def eloop(j, carry):
        chunk = j * 32 + w

        @pl.when(chunk < nchunks)
        def _():
            pltpu.sync_copy(colr_hbm.at[chunk], colbuf)
            for v in range(_HC // 16):
                cv = colbuf[pl.ds(v * 16, 16)]
                plsc.addupdate_scatter(hist, [cv], ones)

        return carry

    lax.fori_loop(0, jmax, eloop, 0)
    pltpu.sync_copy(hist, out_hbm.at[w])


def _hist_stage(col, n):
    e = col.shape[0]
    hc = _HC if e % _HC == 0 else _EC
    colr = col.reshape(e // hc, hc)
    hr = (n + 15) // 16 * 16
    zeros = jnp.zeros((hr,), jnp.float32)
    mesh = plsc.VectorSubcoreMesh(core_axis_name="c", subcore_axis_name="s")
    hists = pl.kernel(
        _hist_body,
        out_type=jax.ShapeDtypeStruct((32, hr), jnp.float32),
        mesh=mesh,
        compiler_params=pltpu.CompilerParams(needs_layout_passes=False),
        scratch_types=[
            pltpu.VMEM((hr,), jnp.float32),
            pltpu.VMEM((hc,), jnp.int32),
        ],
    )(colr, zeros)
    return hists


def _node_body(x_ref, agg_ref, hist_ref, b_ref, n2w1t, n2w1b, n2b1, n2w2, n2b2,
               gw1, gb1, gw2p, gb2p, u_ref, gsum, gcnt, *, nblocks, num_graphs):
    i = pl.program_id(0)

    @pl.when(i == 0)
    def _():
        gsum[...] = jnp.zeros_like(gsum)
        gcnt[...] = jnp.zeros_like(gcnt)

    x = x_ref[...]
    aggs = agg_ref[...]
    cnt = jnp.sum(hist_ref[0], axis=0)
    agg = aggs / jnp.maximum(cnt, 1.0)[:, None]
    h = jnp.maximum(_dot(x, n2w1t[...]) + _dot(agg, n2w1b[...]) + n2b1[...], 0.0)
    x2 = _dot(h, n2w2[...]) + n2b2[...]
    b = b_ref[0, 0, :]
    bn = x.shape[0]
    onehot = (b[:, None] == jax.lax.broadcasted_iota(jnp.int32, (bn, num_graphs), 1)
              ).astype(jnp.float32)
    seg = lambda v: jax.lax.dot_general(onehot, v, (((0,), (0,)), ((), ())),
                                        preferred_element_type=jnp.float32)
    gsum[...] += seg(x2)
    gcnt[...] += seg(jnp.ones_like(x2))

    @pl.when(i == nblocks - 1)
    def _():
        gmean = gsum[...] / jnp.maximum(gcnt[...], 1.0)
        hg = jnp.maximum(_dot(gmean, gw1[...]) + gb1[...], 0.0)
        u_ref[...] = _dot(hg, gw2p[...]) + gb2p[...]


def _node_stage(x, agg, hists, batch, n2w1, n2b1, n2w2, n2b2, gw1, gb1, gw2,
                gb2, block_n, num_graphs):
    n = x.shape[0]
    grid = n // block_n
    batch3d = batch.reshape(grid, 1, block_n)
    hist3d = hists[:, :n].reshape(32, grid, block_n).transpose(1, 0, 2)
    gw2p = jnp.zeros((128, 128), jnp.float32).at[:, :2].set(gw2)
    gb2p = jnp.zeros((1, 128), jnp.float32).at[0, :2].set(gb2)
    full = lambda s: pl.BlockSpec(s, lambda i: (0,) * len(s))
    u_full = pl.pallas_call(
        functools.partial(_node_body, nblocks=grid, num_graphs=num_graphs),
        grid=(grid,),
        in_specs=[
            pl.BlockSpec((block_n, 48), lambda i: (i, 0)),
            pl.BlockSpec((block_n, 128), lambda i: (i, 0)),
            pl.BlockSpec((1, 32, block_n), lambda i: (i, 0, 0)),
            pl.BlockSpec((1, 1, block_n), lambda i: (i, 0, 0)),
            full((48, 128)), full((128, 128)), full((1, 128)),
            full((128, 128)), full((1, 128)),
            full((128, 128)), full((1, 128)), full((128, 128)), full((1, 128)),
        ],
        out_specs=pl.BlockSpec((num_graphs, 128), lambda i: (0, 0)),
        out_shape=jax.ShapeDtypeStruct((num_graphs, 128), jnp.float32),
        scratch_shapes=[
            pltpu.VMEM((num_graphs, 128), jnp.float32),
            pltpu.VMEM((num_graphs, 128), jnp.float32),
        ],
    )(x, agg, hist3d, batch3d, n2w1[:48], n2w1[48:], n2b1.reshape(1, 128),
      n2w2, n2b2.reshape(1, 128), gw1, gb1.reshape(1, 128), gw2p, gb2p)
    return u_full[:, :2]


def kernel(x, edge_index, batch, ew1, eb1, ew2, eb2, n1w1, n1b1, n1w2, n1b2,
           n2w1, n2b1, n2w2, n2b2, gw1, gb1, gw2, gb2):
    n = x.shape[0]
    e = edge_index.shape[1]
    row, col = edge_index[0], edge_index[1]

    x16 = x.astype(jnp.bfloat16)
    xr = jnp.take(x16, row, axis=0)
    xc = jnp.take(x16, col, axis=0)

    block_e = 2000 if e % 2000 == 0 else e
    ea, m = _edge_stage(xr, xc, ew1, eb1, ew2, eb2, n1w1, n1b1, n1w2, n1b2,
                        block_e)

    agg = _scatter_stage(m, col, n)
    hists = _hist_stage(col, n)

    block_n = 2000 if n % 2000 == 0 else n
    u = _node_stage(x, agg, hists, batch, n2w1, n2b1, n2w2, n2b2, gw1, gb1,
                    gw2, gb2, block_n, 64)
    return (u, ea)


# speedup vs baseline: 2.2653x; 1.8377x over previous
import functools

import jax
import jax.numpy as jnp
from jax import lax
from jax.experimental import pallas as pl
from jax.experimental.pallas import tpu as pltpu
from jax.experimental.pallas import tpu_sc as plsc


def _dot(a, b):
    return jax.lax.dot_general(a, b, (((1,), (0,)), ((), ())),
                               preferred_element_type=jnp.float32)


def _edge_body(xr_ref, xc_ref, ew1t, ew1b, eb1, ew2, eb2,
               n1w1t, n1w1b, n1b1, n1w2, n1b2, ea_ref, m_ref):
    bf = jnp.bfloat16
    xr = xr_ref[...]
    xc = xc_ref[...]
    h1 = jnp.maximum(_dot(xr, ew1t[...]) + _dot(xc, ew1b[...]) + eb1[...],
                     0.0)
    ea = _dot(h1, ew2[...]) + eb2[...]
    ea_ref[...] = ea
    h2 = jnp.maximum(_dot(xr.astype(bf), n1w1t[...].astype(bf))
                     + _dot(ea.astype(bf), n1w1b[...].astype(bf)) + n1b1[...], 0.0)
    m_ref[...] = _dot(h2.astype(bf), n1w2[...].astype(bf)) + n1b2[...]


def _edge_stage(xr, xc, ew1, eb1, ew2, eb2, n1w1, n1b1, n1w2, n1b2, block_e):
    e = xr.shape[0]
    grid = e // block_e
    full = lambda s: pl.BlockSpec(s, lambda i: (0, 0))
    ea, m = pl.pallas_call(
        _edge_body,
        grid=(grid,),
        in_specs=[
            pl.BlockSpec((block_e, 128), lambda i: (i, 0)),
            pl.BlockSpec((block_e, 128), lambda i: (i, 0)),
            full((128, 128)), full((128, 128)), full((1, 128)),
            full((128, 128)), full((1, 128)),
            full((128, 128)), full((128, 128)), full((1, 128)),
            full((128, 128)), full((1, 128)),
        ],
        out_specs=[
            pl.BlockSpec((block_e, 128), lambda i: (i, 0)),
            pl.BlockSpec((block_e, 128), lambda i: (i, 0)),
        ],
        out_shape=[
            jax.ShapeDtypeStruct((e, 128), jnp.float32),
            jax.ShapeDtypeStruct((e, 128), jnp.float32),
        ],
    )(xr, xc,
      jnp.zeros((128, 128), jnp.float32).at[:48].set(ew1[:48]),
      jnp.zeros((128, 128), jnp.float32).at[:48].set(ew1[48:]),
      eb1.reshape(1, 128), ew2, eb2.reshape(1, 128),
      jnp.zeros((128, 128), jnp.float32).at[:48].set(n1w1[:48]),
      n1w1[48:], n1b1.reshape(1, 128), n1w2, n1b2.reshape(1, 128))
    return ea, m


_GB = 256


def _gather_body(x_hbm, rowr_hbm, colr_hbm, xr_hbm, xc_hbm,
                 idxr, idxc, gr, gc, semr, semc):
    c = lax.axis_index("c")
    s = lax.axis_index("s")
    w = s * 2 + c
    nsc = rowr_hbm.shape[0]
    jmax = (nsc + 31) // 32

    def gloop(j, carry):
        chunk = j * 32 + w

        @pl.when(chunk < nsc)
        def _():
            pltpu.sync_copy(rowr_hbm.at[chunk], idxr)
            pltpu.sync_copy(colr_hbm.at[chunk], idxc)
            descs = []
            for u in range(_GB // 128):
                descs.append(pltpu.async_copy(
                    x_hbm.at[idxr.at[u]], gr.at[pl.ds(u * 128, 128)], semr))
                descs.append(pltpu.async_copy(
                    x_hbm.at[idxc.at[u]], gc.at[pl.ds(u * 128, 128)], semc))
            for d in descs:
                d.wait()
            pltpu.sync_copy(gr, xr_hbm.at[pl.ds(chunk * _GB, _GB)])
            pltpu.sync_copy(gc, xc_hbm.at[pl.ds(chunk * _GB, _GB)])

        return carry

    lax.fori_loop(0, jmax, gloop, 0)


def _gather_stage(x, row, col):
    e = row.shape[0]
    n = x.shape[0]
    xp = jnp.zeros((n, 128), jnp.float32).at[:, :x.shape[1]].set(x)
    rowr = row.reshape(e // _GB, _GB // 128, 128)
    colr = col.reshape(e // _GB, _GB // 128, 128)
    mesh = plsc.VectorSubcoreMesh(core_axis_name="c", subcore_axis_name="s")
    xr, xc = pl.kernel(
        _gather_body,
        out_type=[jax.ShapeDtypeStruct((e, 128), jnp.float32),
                  jax.ShapeDtypeStruct((e, 128), jnp.float32)],
        mesh=mesh,
        scratch_types=[
            pltpu.VMEM((_GB // 128, 128), jnp.int32),
            pltpu.VMEM((_GB // 128, 128), jnp.int32),
            pltpu.VMEM((_GB, 128), jnp.float32),
            pltpu.VMEM((_GB, 128), jnp.float32),
            pltpu.SemaphoreType.DMA,
            pltpu.SemaphoreType.DMA,
        ],
    )(xp, rowr, colr)
    return xr, xc


_EC = 128


def _scatter_body(m_hbm, colr_hbm, zeros_hbm, agg_hbm, acc, colbuf, libuf, mbuf,
                  *, nch, stripe):
    c = lax.axis_index("c")
    s = lax.axis_index("s")
    nchunks = colr_hbm.shape[0]
    jmax = (nchunks + 15) // 16
    full_tiles = nch // stripe
    partial = nch - full_tiles * stripe
    for p in range(2):
        base = (2 * p + c) * nch
        pltpu.sync_copy(zeros_hbm.at[pl.ds(s * stripe, stripe)],
                        acc.at[pl.ds(s * stripe, stripe)])
        plsc.subcore_barrier()

        def eloop(j, carry):
            chunk = j * 16 + s

            @pl.when(chunk < nchunks)
            def _():
                pltpu.sync_copy(colr_hbm.at[chunk], colbuf)
                pltpu.sync_copy(m_hbm.at[pl.ds(chunk * _EC, _EC)], mbuf)
                for v in range(_EC // 16):
                    cv = colbuf[pl.ds(v * 16, 16)]
                    li = cv - base
                    memb = (li >= 0) & (li < nch)
                    libuf[pl.ds(v * 16, 16)] = jnp.where(memb, li, nch)
                pltpu.sync_copy(mbuf, acc.at[libuf], add=True)

            return carry

        lax.fori_loop(0, jmax, eloop, 0)
        plsc.subcore_barrier()
        start = s * stripe

        @pl.when(s < full_tiles)
        def _():
            pltpu.sync_copy(acc.at[pl.ds(start, stripe)],
                            agg_hbm.at[pl.ds(base + start, stripe)])

        if partial:
            @pl.when(s == full_tiles)
            def _():
                pltpu.sync_copy(acc.at[pl.ds(start, partial)],
                                agg_hbm.at[pl.ds(base + start, partial)])

        plsc.subcore_barrier()


def _scatter_stage(m, col, n):
    e = m.shape[0]
    colr = col.reshape(e // _EC, _EC)
    nch = (-(-n // 4) + 7) // 8 * 8
    stripe = (-(-(nch + 1) // 16) + 7) // 8 * 8
    accrows = stripe * 16
    zeros = jnp.zeros((accrows, 128), jnp.float32)
    mesh = plsc.VectorSubcoreMesh(core_axis_name="c", subcore_axis_name="s")
    agg = pl.kernel(
        functools.partial(_scatter_body, nch=nch, stripe=stripe),
        out_type=jax.ShapeDtypeStruct((4 * nch, 128), jnp.float32),
        mesh=mesh,
        scratch_types=[
            pltpu.VMEM_SHARED((accrows, 128), jnp.float32),
            pltpu.VMEM((_EC,), jnp.int32),
            pltpu.VMEM((_EC,), jnp.int32),
            pltpu.VMEM((_EC, 128), jnp.float32),
        ],
    )(m, colr, zeros)
    return agg


_HC = 1600


def _hist_body(colr_hbm, zeros_hbm, out_hbm, hist, colbuf):
    c = lax.axis_index("c")
    s = lax.axis_index("s")
    w = s * 2 + c
    nchunks = colr_hbm.shape[0]
    jmax = (nchunks + 31) // 32
    pltpu.sync_copy(zeros_hbm, hist)
    ones = jnp.ones((16,), jnp.float32)

    def eloop(j, carry):
        chunk = j * 32 + w

        @pl.when(chunk < nchunks)
        def _():
            pltpu.sync_copy(colr_hbm.at[chunk], colbuf)
            for v in range(_HC // 16):
                cv = colbuf[pl.ds(v * 16, 16)]
                plsc.addupdate_scatter(hist, [cv], ones)

        return carry

    lax.fori_loop(0, jmax, eloop, 0)
    pltpu.sync_copy(hist, out_hbm.at[w])


def _hist_stage(col, n):
    e = col.shape[0]
    hc = _HC if e % _HC == 0 else _EC
    colr = col.reshape(e // hc, hc)
    hr = (n + 15) // 16 * 16
    zeros = jnp.zeros((hr,), jnp.float32)
    mesh = plsc.VectorSubcoreMesh(core_axis_name="c", subcore_axis_name="s")
    hists = pl.kernel(
        _hist_body,
        out_type=jax.ShapeDtypeStruct((32, hr), jnp.float32),
        mesh=mesh,
        compiler_params=pltpu.CompilerParams(needs_layout_passes=False),
        scratch_types=[
            pltpu.VMEM((hr,), jnp.float32),
            pltpu.VMEM((hc,), jnp.int32),
        ],
    )(colr, zeros)
    return hists


def _node_body(x_ref, agg_ref, hist_ref, b_ref, n2w1t, n2w1b, n2b1, n2w2, n2b2,
               gw1, gb1, gw2p, gb2p, u_ref, gsum, gcnt, *, nblocks, num_graphs):
    i = pl.program_id(0)

    @pl.when(i == 0)
    def _():
        gsum[...] = jnp.zeros_like(gsum)
        gcnt[...] = jnp.zeros_like(gcnt)

    x = x_ref[...]
    aggs = agg_ref[...]
    cnt = jnp.sum(hist_ref[0], axis=0)
    agg = aggs / jnp.maximum(cnt, 1.0)[:, None]
    h = jnp.maximum(_dot(x, n2w1t[...]) + _dot(agg, n2w1b[...]) + n2b1[...], 0.0)
    x2 = _dot(h, n2w2[...]) + n2b2[...]
    b = b_ref[0, 0, :]
    bn = x.shape[0]
    onehot = (b[:, None] == jax.lax.broadcasted_iota(jnp.int32, (bn, num_graphs), 1)
              ).astype(jnp.float32)
    seg = lambda v: jax.lax.dot_general(onehot, v, (((0,), (0,)), ((), ())),
                                        preferred_element_type=jnp.float32)
    gsum[...] += seg(x2)
    gcnt[...] += seg(jnp.ones_like(x2))

    @pl.when(i == nblocks - 1)
    def _():
        gmean = gsum[...] / jnp.maximum(gcnt[...], 1.0)
        hg = jnp.maximum(_dot(gmean, gw1[...]) + gb1[...], 0.0)
        u_ref[...] = _dot(hg, gw2p[...]) + gb2p[...]


def _node_stage(x, agg, hists, batch, n2w1, n2b1, n2w2, n2b2, gw1, gb1, gw2,
                gb2, block_n, num_graphs):
    n = x.shape[0]
    grid = n // block_n
    batch3d = batch.reshape(grid, 1, block_n)
    hist3d = hists[:, :n].reshape(32, grid, block_n).transpose(1, 0, 2)
    gw2p = jnp.zeros((128, 128), jnp.float32).at[:, :2].set(gw2)
    gb2p = jnp.zeros((1, 128), jnp.float32).at[0, :2].set(gb2)
    full = lambda s: pl.BlockSpec(s, lambda i: (0,) * len(s))
    u_full = pl.pallas_call(
        functools.partial(_node_body, nblocks=grid, num_graphs=num_graphs),
        grid=(grid,),
        in_specs=[
            pl.BlockSpec((block_n, 48), lambda i: (i, 0)),
            pl.BlockSpec((block_n, 128), lambda i: (i, 0)),
            pl.BlockSpec((1, 32, block_n), lambda i: (i, 0, 0)),
            pl.BlockSpec((1, 1, block_n), lambda i: (i, 0, 0)),
            full((48, 128)), full((128, 128)), full((1, 128)),
            full((128, 128)), full((1, 128)),
            full((128, 128)), full((1, 128)), full((128, 128)), full((1, 128)),
        ],
        out_specs=pl.BlockSpec((num_graphs, 128), lambda i: (0, 0)),
        out_shape=jax.ShapeDtypeStruct((num_graphs, 128), jnp.float32),
        scratch_shapes=[
            pltpu.VMEM((num_graphs, 128), jnp.float32),
            pltpu.VMEM((num_graphs, 128), jnp.float32),
        ],
    )(x, agg, hist3d, batch3d, n2w1[:48], n2w1[48:], n2b1.reshape(1, 128),
      n2w2, n2b2.reshape(1, 128), gw1, gb1.reshape(1, 128), gw2p, gb2p)
    return u_full[:, :2]


def kernel(x, edge_index, batch, ew1, eb1, ew2, eb2, n1w1, n1b1, n1w2, n1b2,
           n2w1, n2b1, n2w2, n2b2, gw1, gb1, gw2, gb2):
    n = x.shape[0]
    e = edge_index.shape[1]
    row, col = edge_index[0], edge_index[1]

    if e % _GB == 0:
        xr, xc = _gather_stage(x, row, col)
    else:
        xp = jnp.zeros((n, 128), jnp.float32).at[:, :48].set(x)
        xr = jnp.take(xp, row, axis=0)
        xc = jnp.take(xp, col, axis=0)

    block_e = 2000 if e % 2000 == 0 else e
    ea, m = _edge_stage(xr, xc, ew1, eb1, ew2, eb2, n1w1, n1b1, n1w2, n1b2,
                        block_e)

    agg = _scatter_stage(m, col, n)
    hists = _hist_stage(col, n)

    block_n = 2000 if n % 2000 == 0 else n
    u = _node_stage(x, agg, hists, batch, n2w1, n2b1, n2w2, n2b2, gw1, gb1,
                    gw2, gb2, block_n, 64)
    return (u, ea)


# speedup vs baseline: 2.5522x; 1.1267x over previous
import functools

import jax
import jax.numpy as jnp
from jax import lax
from jax.experimental import pallas as pl
from jax.experimental.pallas import tpu as pltpu
from jax.experimental.pallas import tpu_sc as plsc


def _dot(a, b):
    return jax.lax.dot_general(a, b, (((1,), (0,)), ((), ())),
                               preferred_element_type=jnp.float32)


def _edge_body(xr_ref, xc_ref, ew1t, ew1b, eb1, ew2, eb2,
               n1w1t, n1w1b, n1b1, n1w2, n1b2, ea_ref, m_ref):
    bf = jnp.bfloat16
    xr = xr_ref[...]
    xc = xc_ref[...]
    h1 = jnp.maximum(_dot(xr, ew1t[...]) + _dot(xc, ew1b[...]) + eb1[...],
                     0.0)
    ea = _dot(h1, ew2[...]) + eb2[...]
    ea_ref[...] = ea
    h2 = jnp.maximum(_dot(xr.astype(bf), n1w1t[...].astype(bf))
                     + _dot(ea.astype(bf), n1w1b[...].astype(bf)) + n1b1[...], 0.0)
    m_ref[...] = _dot(h2.astype(bf), n1w2[...].astype(bf)) + n1b2[...]


def _edge_stage(xr, xc, ew1, eb1, ew2, eb2, n1w1, n1b1, n1w2, n1b2, block_e):
    e = xr.shape[0]
    grid = e // block_e
    full = lambda s: pl.BlockSpec(s, lambda i: (0, 0))
    ea, m = pl.pallas_call(
        _edge_body,
        grid=(grid,),
        in_specs=[
            pl.BlockSpec((block_e, 128), lambda i: (i, 0)),
            pl.BlockSpec((block_e, 128), lambda i: (i, 0)),
            full((128, 128)), full((128, 128)), full((1, 128)),
            full((128, 128)), full((1, 128)),
            full((128, 128)), full((128, 128)), full((1, 128)),
            full((128, 128)), full((1, 128)),
        ],
        out_specs=[
            pl.BlockSpec((block_e, 128), lambda i: (i, 0)),
            pl.BlockSpec((block_e, 128), lambda i: (i, 0)),
        ],
        out_shape=[
            jax.ShapeDtypeStruct((e, 128), jnp.float32),
            jax.ShapeDtypeStruct((e, 128), jnp.float32),
        ],
    )(xr, xc,
      jnp.zeros((128, 128), jnp.float32).at[:48].set(ew1[:48]),
      jnp.zeros((128, 128), jnp.float32).at[:48].set(ew1[48:]),
      eb1.reshape(1, 128), ew2, eb2.reshape(1, 128),
      jnp.zeros((128, 128), jnp.float32).at[:48].set(n1w1[:48]),
      n1w1[48:], n1b1.reshape(1, 128), n1w2, n1b2.reshape(1, 128))
    return ea, m


_GB = 256


def _gather_body(x_hbm, rowr_hbm, colr_hbm, xr_hbm, xc_hbm,
                 idxr, idxc, gr, gc, semr, semc):
    c = lax.axis_index("c")
    s = lax.axis_index("s")
    w = s * 2 + c
    nsc = rowr_hbm.shape[0]
    jmax = (nsc + 31) // 32

    def gloop(j, carry):
        chunk = j * 32 + w

        @pl.when(chunk < nsc)
        def _():
            pltpu.sync_copy(rowr_hbm.at[chunk], idxr)
            pltpu.sync_copy(colr_hbm.at[chunk], idxc)
            descs = []
            for u in range(_GB // 128):
                descs.append(pltpu.async_copy(
                    x_hbm.at[idxr.at[u]], gr.at[pl.ds(u * 128, 128)], semr))
                descs.append(pltpu.async_copy(
                    x_hbm.at[idxc.at[u]], gc.at[pl.ds(u * 128, 128)], semc))
            for d in descs:
                d.wait()
            pltpu.sync_copy(gr, xr_hbm.at[pl.ds(chunk * _GB, _GB)])
            pltpu.sync_copy(gc, xc_hbm.at[pl.ds(chunk * _GB, _GB)])

        return carry

    lax.fori_loop(0, jmax, gloop, 0)


def _gather_stage(x, row, col):
    e = row.shape[0]
    n = x.shape[0]
    xp = jnp.zeros((n, 128), jnp.float32).at[:, :x.shape[1]].set(x)
    rowr = row.reshape(e // _GB, _GB // 128, 128)
    colr = col.reshape(e // _GB, _GB // 128, 128)
    mesh = plsc.VectorSubcoreMesh(core_axis_name="c", subcore_axis_name="s")
    xr, xc = pl.kernel(
        _gather_body,
        out_type=[jax.ShapeDtypeStruct((e, 128), jnp.float32),
                  jax.ShapeDtypeStruct((e, 128), jnp.float32)],
        mesh=mesh,
        scratch_types=[
            pltpu.VMEM((_GB // 128, 128), jnp.int32),
            pltpu.VMEM((_GB // 128, 128), jnp.int32),
            pltpu.VMEM((_GB, 128), jnp.float32),
            pltpu.VMEM((_GB, 128), jnp.float32),
            pltpu.SemaphoreType.DMA,
            pltpu.SemaphoreType.DMA,
        ],
    )(xp, rowr, colr)
    return xr, xc


_EC = 64


_SLOTS = 3


def _scatter_body(m_hbm, colr_hbm, zeros_hbm, agg_hbm, acc, colbuf, libuf, mbuf,
                  sg0, sg1, sg2, sa0, sa1, sa2, *, nch, stripe):
    c = lax.axis_index("c")
    s = lax.axis_index("s")
    nchunks = colr_hbm.shape[0]
    nbt = (nchunks + 15) // 16
    full_tiles = nch // stripe
    partial = nch - full_tiles * stripe
    sgs = [sg0, sg1, sg2]
    sas = [sa0, sa1, sa2]

    def valid(b):
        return (b >= 0) & (b < nbt) & (s * nbt + b < nchunks)

    def fire_stage(b, u):
        chunk = s * nbt + b

        @pl.when(valid(b))
        def _():
            pltpu.async_copy(colr_hbm.at[chunk], colbuf.at[u], sgs[u])
            pltpu.async_copy(m_hbm.at[pl.ds(chunk * _EC, _EC)],
                             mbuf.at[pl.ds(u * _EC, _EC)], sgs[u])

    def wait_stage(b, u):
        @pl.when(valid(b))
        def _():
            pltpu.make_async_copy(colr_hbm.at[0], colbuf.at[u], sgs[u]).wait()
            pltpu.make_async_copy(m_hbm.at[pl.ds(0, _EC)],
                                  mbuf.at[pl.ds(u * _EC, _EC)], sgs[u]).wait()

    def fire_add(b, u, base):
        @pl.when(valid(b))
        def _():
            for v in range(_EC // 16):
                cv = colbuf[u, pl.ds(v * 16, 16)]
                li = cv - base
                memb = (li >= 0) & (li < nch)
                libuf[u, pl.ds(v * 16, 16)] = jnp.where(memb, li, nch)
            pltpu.async_copy(mbuf.at[pl.ds(u * _EC, _EC)], acc.at[libuf.at[u]],
                             sas[u], add=True)

    def wait_add(b, u):
        @pl.when(valid(b))
        def _():
            pltpu.make_async_copy(m_hbm.at[pl.ds(0, _EC)],
                                  mbuf.at[pl.ds(u * _EC, _EC)], sas[u]).wait()

    nbody = (nbt + _SLOTS - 1) // _SLOTS
    for p in range(2):
        base = (2 * p + c) * nch
        pltpu.sync_copy(zeros_hbm.at[pl.ds(s * stripe, stripe)],
                        acc.at[pl.ds(s * stripe, stripe)])
        plsc.subcore_barrier()

        fire_stage(0, 0)
        fire_stage(1, 1)
        for b in range(_SLOTS):
            if b >= 1:
                wait_add(b - 1, (b - 1) % _SLOTS)
            fire_stage(b + 2, (b + 2) % _SLOTS)
            wait_stage(b, b % _SLOTS)
            fire_add(b, b % _SLOTS, base)

        def body(t, carry):
            for u in range(_SLOTS):
                b = t * _SLOTS + u
                wait_add(b - 1, (u + 2) % _SLOTS)
                fire_stage(b + 2, (u + 2) % _SLOTS)
                wait_stage(b, u)
                fire_add(b, u, base)
            return carry

        lax.fori_loop(1, nbody, body, 0)
        for b in range(_SLOTS * nbody - 1, nbt):
            wait_add(b, b % _SLOTS)
        plsc.subcore_barrier()
        start = s * stripe

        @pl.when(s < full_tiles)
        def _():
            pltpu.sync_copy(acc.at[pl.ds(start, stripe)],
                            agg_hbm.at[pl.ds(base + start, stripe)])

        if partial:
            @pl.when(s == full_tiles)
            def _():
                pltpu.sync_copy(acc.at[pl.ds(start, partial)],
                                agg_hbm.at[pl.ds(base + start, partial)])

        plsc.subcore_barrier()


def _scatter_stage(m, col, n):
    e = m.shape[0]
    colr = col.reshape(e // _EC, _EC)
    nch = (-(-n // 4) + 7) // 8 * 8
    stripe = (-(-(nch + 1) // 16) + 7) // 8 * 8
    accrows = stripe * 16
    zeros = jnp.zeros((accrows, 128), jnp.float32)
    mesh = plsc.VectorSubcoreMesh(core_axis_name="c", subcore_axis_name="s")
    agg = pl.kernel(
        functools.partial(_scatter_body, nch=nch, stripe=stripe),
        out_type=jax.ShapeDtypeStruct((4 * nch, 128), jnp.float32),
        mesh=mesh,
        scratch_types=[
            pltpu.VMEM_SHARED((accrows, 128), jnp.float32),
            pltpu.VMEM((_SLOTS, _EC), jnp.int32),
            pltpu.VMEM((_SLOTS, _EC), jnp.int32),
            pltpu.VMEM((_SLOTS * _EC, 128), jnp.float32),
        ] + [pltpu.SemaphoreType.DMA] * 6,
    )(m, colr, zeros)
    return agg


_HC = 1600


def _hist_body(colr_hbm, zeros_hbm, out_hbm, hist, colbuf):
    c = lax.axis_index("c")
    s = lax.axis_index("s")
    w = s * 2 + c
    nchunks = colr_hbm.shape[0]
    jmax = (nchunks + 31) // 32
    pltpu.sync_copy(zeros_hbm, hist)
    ones = jnp.ones((16,), jnp.float32)

    def eloop(j, carry):
        chunk = j * 32 + w

        @pl.when(chunk < nchunks)
        def _():
            pltpu.sync_copy(colr_hbm.at[chunk], colbuf)
            for v in range(_HC // 16):
                cv = colbuf[pl.ds(v * 16, 16)]
                plsc.addupdate_scatter(hist, [cv], ones)

        return carry

    lax.fori_loop(0, jmax, eloop, 0)
    pltpu.sync_copy(hist, out_hbm.at[w])


def _hist_stage(col, n):
    e = col.shape[0]
    hc = _HC if e % _HC == 0 else _EC
    colr = col.reshape(e // hc, hc)
    hr = (n + 15) // 16 * 16
    zeros = jnp.zeros((hr,), jnp.float32)
    mesh = plsc.VectorSubcoreMesh(core_axis_name="c", subcore_axis_name="s")
    hists = pl.kernel(
        _hist_body,
        out_type=jax.ShapeDtypeStruct((32, hr), jnp.float32),
        mesh=mesh,
        compiler_params=pltpu.CompilerParams(needs_layout_passes=False),
        scratch_types=[
            pltpu.VMEM((hr,), jnp.float32),
            pltpu.VMEM((hc,), jnp.int32),
        ],
    )(colr, zeros)
    return hists


def _node_body(x_ref, agg_ref, hist_ref, b_ref, n2w1t, n2w1b, n2b1, n2w2, n2b2,
               gw1, gb1, gw2p, gb2p, u_ref, gsum, gcnt, *, nblocks, num_graphs):
    i = pl.program_id(0)

    @pl.when(i == 0)
    def _():
        gsum[...] = jnp.zeros_like(gsum)
        gcnt[...] = jnp.zeros_like(gcnt)

    x = x_ref[...]
    aggs = agg_ref[...]
    cnt = jnp.sum(hist_ref[0], axis=0)
    agg = aggs / jnp.maximum(cnt, 1.0)[:, None]
    h = jnp.maximum(_dot(x, n2w1t[...]) + _dot(agg, n2w1b[...]) + n2b1[...], 0.0)
    x2 = _dot(h, n2w2[...]) + n2b2[...]
    b = b_ref[0, 0, :]
    bn = x.shape[0]
    onehot = (b[:, None] == jax.lax.broadcasted_iota(jnp.int32, (bn, num_graphs), 1)
              ).astype(jnp.float32)
    seg = lambda v: jax.lax.dot_general(onehot, v, (((0,), (0,)), ((), ())),
                                        preferred_element_type=jnp.float32)
    gsum[...] += seg(x2)
    gcnt[...] += seg(jnp.ones_like(x2))

    @pl.when(i == nblocks - 1)
    def _():
        gmean = gsum[...] / jnp.maximum(gcnt[...], 1.0)
        hg = jnp.maximum(_dot(gmean, gw1[...]) + gb1[...], 0.0)
        u_ref[...] = _dot(hg, gw2p[...]) + gb2p[...]


def _node_stage(x, agg, hists, batch, n2w1, n2b1, n2w2, n2b2, gw1, gb1, gw2,
                gb2, block_n, num_graphs):
    n = x.shape[0]
    grid = n // block_n
    batch3d = batch.reshape(grid, 1, block_n)
    hist3d = hists[:, :n].reshape(32, grid, block_n).transpose(1, 0, 2)
    gw2p = jnp.zeros((128, 128), jnp.float32).at[:, :2].set(gw2)
    gb2p = jnp.zeros((1, 128), jnp.float32).at[0, :2].set(gb2)
    full = lambda s: pl.BlockSpec(s, lambda i: (0,) * len(s))
    u_full = pl.pallas_call(
        functools.partial(_node_body, nblocks=grid, num_graphs=num_graphs),
        grid=(grid,),
        in_specs=[
            pl.BlockSpec((block_n, 48), lambda i: (i, 0)),
            pl.BlockSpec((block_n, 128), lambda i: (i, 0)),
            pl.BlockSpec((1, 32, block_n), lambda i: (i, 0, 0)),
            pl.BlockSpec((1, 1, block_n), lambda i: (i, 0, 0)),
            full((48, 128)), full((128, 128)), full((1, 128)),
            full((128, 128)), full((1, 128)),
            full((128, 128)), full((1, 128)), full((128, 128)), full((1, 128)),
        ],
        out_specs=pl.BlockSpec((num_graphs, 128), lambda i: (0, 0)),
        out_shape=jax.ShapeDtypeStruct((num_graphs, 128), jnp.float32),
        scratch_shapes=[
            pltpu.VMEM((num_graphs, 128), jnp.float32),
            pltpu.VMEM((num_graphs, 128), jnp.float32),
        ],
    )(x, agg, hist3d, batch3d, n2w1[:48], n2w1[48:], n2b1.reshape(1, 128),
      n2w2, n2b2.reshape(1, 128), gw1, gb1.reshape(1, 128), gw2p, gb2p)
    return u_full[:, :2]


def kernel(x, edge_index, batch, ew1, eb1, ew2, eb2, n1w1, n1b1, n1w2, n1b2,
           n2w1, n2b1, n2w2, n2b2, gw1, gb1, gw2, gb2):
    n = x.shape[0]
    e = edge_index.shape[1]
    row, col = edge_index[0], edge_index[1]

    if e % _GB == 0:
        xr, xc = _gather_stage(x, row, col)
    else:
        xp = jnp.zeros((n, 128), jnp.float32).at[:, :48].set(x)
        xr = jnp.take(xp, row, axis=0)
        xc = jnp.take(xp, col, axis=0)

    block_e = 2000 if e % 2000 == 0 else e
    ea, m = _edge_stage(xr, xc, ew1, eb1, ew2, eb2, n1w1, n1b1, n1w2, n1b2,
                        block_e)

    agg = _scatter_stage(m, col, n)
    hists = _hist_stage(col, n)

    block_n = 2000 if n % 2000 == 0 else n
    u = _node_stage(x, agg, hists, batch, n2w1, n2b1, n2w2, n2b2, gw1, gb1,
                    gw2, gb2, block_n, 64)
    return (u, ea)


# speedup vs baseline: 2.6780x; 1.0493x over previous
import functools

import jax
import jax.numpy as jnp
from jax import lax
from jax.experimental import pallas as pl
from jax.experimental.pallas import tpu as pltpu
from jax.experimental.pallas import tpu_sc as plsc


def _dot(a, b):
    return jax.lax.dot_general(a, b, (((1,), (0,)), ((), ())),
                               preferred_element_type=jnp.float32)


def _edge_body(xr_ref, xc_ref, ew1t, ew1b, eb1, ew2, eb2,
               n1w1t, n1w1b, n1b1, n1w2, n1b2, ea_ref, m_ref):
    bf = jnp.bfloat16
    xr = xr_ref[...]
    xc = xc_ref[...]
    h1 = jnp.maximum(_dot(xr, ew1t[...]) + _dot(xc, ew1b[...]) + eb1[...],
                     0.0)
    ea = _dot(h1, ew2[...]) + eb2[...]
    ea_ref[...] = ea
    h2 = jnp.maximum(_dot(xr.astype(bf), n1w1t[...].astype(bf))
                     + _dot(ea.astype(bf), n1w1b[...].astype(bf)) + n1b1[...], 0.0)
    m_ref[...] = _dot(h2.astype(bf), n1w2[...].astype(bf)) + n1b2[...]


def _edge_stage(xr, xc, ew1, eb1, ew2, eb2, n1w1, n1b1, n1w2, n1b2, block_e):
    e = xr.shape[0]
    grid = e // block_e
    full = lambda s: pl.BlockSpec(s, lambda i: (0, 0))
    ea, m = pl.pallas_call(
        _edge_body,
        grid=(grid,),
        in_specs=[
            pl.BlockSpec((block_e, 128), lambda i: (i, 0)),
            pl.BlockSpec((block_e, 128), lambda i: (i, 0)),
            full((128, 128)), full((128, 128)), full((1, 128)),
            full((128, 128)), full((1, 128)),
            full((128, 128)), full((128, 128)), full((1, 128)),
            full((128, 128)), full((1, 128)),
        ],
        out_specs=[
            pl.BlockSpec((block_e, 128), lambda i: (i, 0)),
            pl.BlockSpec((block_e, 128), lambda i: (i, 0)),
        ],
        out_shape=[
            jax.ShapeDtypeStruct((e, 128), jnp.float32),
            jax.ShapeDtypeStruct((e, 128), jnp.float32),
        ],
    )(xr, xc,
      jnp.zeros((128, 128), jnp.float32).at[:48].set(ew1[:48]),
      jnp.zeros((128, 128), jnp.float32).at[:48].set(ew1[48:]),
      eb1.reshape(1, 128), ew2, eb2.reshape(1, 128),
      jnp.zeros((128, 128), jnp.float32).at[:48].set(n1w1[:48]),
      n1w1[48:], n1b1.reshape(1, 128), n1w2, n1b2.reshape(1, 128))
    return ea, m


_GB = 128


def _gather_body(x_hbm, rowr_hbm, colr_hbm, xr_hbm, xc_hbm,
                 idxr, idxc, gr, gc, si0, si1, si2, si3, sg0, sg1, sw0, sw1):
    c = lax.axis_index("c")
    s = lax.axis_index("s")
    w = s * 2 + c
    nsc = rowr_hbm.shape[0]
    nbt = (nsc + 31) // 32
    sis = [si0, si1, si2, si3]
    sgs = [sg0, sg1]
    sws = [sw0, sw1]

    def valid(b):
        return (b >= 0) & (b < nbt) & (b * 32 + w < nsc)

    def fire_idx(b, v):
        @pl.when(valid(b))
        def _():
            chunk = b * 32 + w
            pltpu.async_copy(rowr_hbm.at[chunk], idxr.at[v], sis[v])
            pltpu.async_copy(colr_hbm.at[chunk], idxc.at[v], sis[v])

    def wait_idx(b, v):
        @pl.when(valid(b))
        def _():
            pltpu.make_async_copy(rowr_hbm.at[0], idxr.at[v], sis[v]).wait()
            pltpu.make_async_copy(colr_hbm.at[0], idxc.at[v], sis[v]).wait()

    def fire_gathers(b, u, v):
        @pl.when(valid(b))
        def _():
            pltpu.async_copy(x_hbm.at[idxr.at[v]],
                             gr.at[pl.ds(u * _GB, _GB)], sgs[u])
            pltpu.async_copy(x_hbm.at[idxc.at[v]],
                             gc.at[pl.ds(u * _GB, _GB)], sgs[u])

    def wait_gathers(b, u):
        @pl.when(valid(b))
        def _():
            pltpu.make_async_copy(x_hbm.at[idxr.at[0]],
                                  gr.at[pl.ds(u * _GB, _GB)], sgs[u]).wait()
            pltpu.make_async_copy(x_hbm.at[idxc.at[0]],
                                  gc.at[pl.ds(u * _GB, _GB)], sgs[u]).wait()

    def fire_write(b, u):
        @pl.when(valid(b))
        def _():
            chunk = b * 32 + w
            pltpu.async_copy(gr.at[pl.ds(u * _GB, _GB)],
                             xr_hbm.at[pl.ds(chunk * _GB, _GB)], sws[u])
            pltpu.async_copy(gc.at[pl.ds(u * _GB, _GB)],
                             xc_hbm.at[pl.ds(chunk * _GB, _GB)], sws[u])

    def wait_write(b, u):
        @pl.when(valid(b))
        def _():
            pltpu.make_async_copy(gr.at[pl.ds(u * _GB, _GB)],
                                  xr_hbm.at[pl.ds(0, _GB)], sws[u]).wait()
            pltpu.make_async_copy(gc.at[pl.ds(u * _GB, _GB)],
                                  xc_hbm.at[pl.ds(0, _GB)], sws[u]).wait()

    def step(b):
        if b >= 1:
            wait_gathers(b - 1, (b - 1) % 2)
            fire_write(b - 1, (b - 1) % 2)
        if b >= 2:
            wait_write(b - 2, b % 2)
        wait_idx(b, b % 4)
        fire_gathers(b, b % 2, b % 4)
        fire_idx(b + 2, (b + 2) % 4)

    fire_idx(0, 0)
    fire_idx(1, 1)
    for b in range(4):
        step(b)

    def body(t, carry):
        for u in range(4):
            b = t * 4 + u
            wait_gathers(b - 1, (u + 1) % 2)
            fire_write(b - 1, (u + 1) % 2)
            wait_write(b - 2, u % 2)
            wait_idx(b, u)
            fire_gathers(b, u % 2, u)
            fire_idx(b + 2, (u + 2) % 4)
        return carry

    lax.fori_loop(1, (nbt + 3) // 4, body, 0)
    last = ((nbt + 3) // 4) * 4 - 1
    wait_gathers(last, last % 2)
    fire_write(last, last % 2)
    wait_write(last - 1, (last - 1) % 2)
    wait_write(last, last % 2)


def _gather_stage(x, row, col):
    e = row.shape[0]
    n = x.shape[0]
    xp = jnp.zeros((n, 128), jnp.float32).at[:, :x.shape[1]].set(x)
    rowr = row.reshape(e // _GB, _GB)
    colr = col.reshape(e // _GB, _GB)
    mesh = plsc.VectorSubcoreMesh(core_axis_name="c", subcore_axis_name="s")
    xr, xc = pl.kernel(
        _gather_body,
        out_type=[jax.ShapeDtypeStruct((e, 128), jnp.float32),
                  jax.ShapeDtypeStruct((e, 128), jnp.float32)],
        mesh=mesh,
        scratch_types=[
            pltpu.VMEM((4, _GB), jnp.int32),
            pltpu.VMEM((4, _GB), jnp.int32),
            pltpu.VMEM((2 * _GB, 128), jnp.float32),
            pltpu.VMEM((2 * _GB, 128), jnp.float32),
        ] + [pltpu.SemaphoreType.DMA] * 8,
    )(xp, rowr, colr)
    return xr, xc


_EC = 64


_SLOTS = 3


def _scatter_body(m_hbm, colr_hbm, zeros_hbm, agg_hbm, acc, colbuf, libuf, mbuf,
                  sg0, sg1, sg2, sa0, sa1, sa2, *, nch, stripe):
    c = lax.axis_index("c")
    s = lax.axis_index("s")
    nchunks = colr_hbm.shape[0]
    nbt = (nchunks + 15) // 16
    full_tiles = nch // stripe
    partial = nch - full_tiles * stripe
    sgs = [sg0, sg1, sg2]
    sas = [sa0, sa1, sa2]

    def valid(b):
        return (b >= 0) & (b < nbt) & (s * nbt + b < nchunks)

    def fire_stage(b, u):
        chunk = s * nbt + b

        @pl.when(valid(b))
        def _():
            pltpu.async_copy(colr_hbm.at[chunk], colbuf.at[u], sgs[u])
            pltpu.async_copy(m_hbm.at[pl.ds(chunk * _EC, _EC)],
                             mbuf.at[pl.ds(u * _EC, _EC)], sgs[u])

    def wait_stage(b, u):
        @pl.when(valid(b))
        def _():
            pltpu.make_async_copy(colr_hbm.at[0], colbuf.at[u], sgs[u]).wait()
            pltpu.make_async_copy(m_hbm.at[pl.ds(0, _EC)],
                                  mbuf.at[pl.ds(u * _EC, _EC)], sgs[u]).wait()

    def fire_add(b, u, base):
        @pl.when(valid(b))
        def _():
            for v in range(_EC // 16):
                cv = colbuf[u, pl.ds(v * 16, 16)]
                li = cv - base
                memb = (li >= 0) & (li < nch)
                libuf[u, pl.ds(v * 16, 16)] = jnp.where(memb, li, nch)
            pltpu.async_copy(mbuf.at[pl.ds(u * _EC, _EC)], acc.at[libuf.at[u]],
                             sas[u], add=True)

    def wait_add(b, u):
        @pl.when(valid(b))
        def _():
            pltpu.make_async_copy(m_hbm.at[pl.ds(0, _EC)],
                                  mbuf.at[pl.ds(u * _EC, _EC)], sas[u]).wait()

    nbody = (nbt + _SLOTS - 1) // _SLOTS
    for p in range(2):
        base = (2 * p + c) * nch
        pltpu.sync_copy(zeros_hbm.at[pl.ds(s * stripe, stripe)],
                        acc.at[pl.ds(s * stripe, stripe)])
        plsc.subcore_barrier()

        fire_stage(0, 0)
        fire_stage(1, 1)
        for b in range(_SLOTS):
            if b >= 1:
                wait_add(b - 1, (b - 1) % _SLOTS)
            fire_stage(b + 2, (b + 2) % _SLOTS)
            wait_stage(b, b % _SLOTS)
            fire_add(b, b % _SLOTS, base)

        def body(t, carry):
            for u in range(_SLOTS):
                b = t * _SLOTS + u
                wait_add(b - 1, (u + 2) % _SLOTS)
                fire_stage(b + 2, (u + 2) % _SLOTS)
                wait_stage(b, u)
                fire_add(b, u, base)
            return carry

        lax.fori_loop(1, nbody, body, 0)
        for b in range(_SLOTS * nbody - 1, nbt):
            wait_add(b, b % _SLOTS)
        plsc.subcore_barrier()
        start = s * stripe

        @pl.when(s < full_tiles)
        def _():
            pltpu.sync_copy(acc.at[pl.ds(start, stripe)],
                            agg_hbm.at[pl.ds(base + start, stripe)])

        if partial:
            @pl.when(s == full_tiles)
            def _():
                pltpu.sync_copy(acc.at[pl.ds(start, partial)],
                                agg_hbm.at[pl.ds(base + start, partial)])

        plsc.subcore_barrier()


def _scatter_stage(m, col, n):
    e = m.shape[0]
    colr = col.reshape(e // _EC, _EC)
    nch = (-(-n // 4) + 7) // 8 * 8
    stripe = (-(-(nch + 1) // 16) + 7) // 8 * 8
    accrows = stripe * 16
    zeros = jnp.zeros((accrows, 128), jnp.float32)
    mesh = plsc.VectorSubcoreMesh(core_axis_name="c", subcore_axis_name="s")
    agg = pl.kernel(
        functools.partial(_scatter_body, nch=nch, stripe=stripe),
        out_type=jax.ShapeDtypeStruct((4 * nch, 128), jnp.float32),
        mesh=mesh,
        scratch_types=[
            pltpu.VMEM_SHARED((accrows, 128), jnp.float32),
            pltpu.VMEM((_SLOTS, _EC), jnp.int32),
            pltpu.VMEM((_SLOTS, _EC), jnp.int32),
            pltpu.VMEM((_SLOTS * _EC, 128), jnp.float32),
        ] + [pltpu.SemaphoreType.DMA] * 6,
    )(m, colr, zeros)
    return agg


_HC = 1600


def _hist_body(colr_hbm, zeros_hbm, out_hbm, hist, colbuf):
    c = lax.axis_index("c")
    s = lax.axis_index("s")
    w = s * 2 + c
    nchunks = colr_hbm.shape[0]
    jmax = (nchunks + 31) // 32
    pltpu.sync_copy(zeros_hbm, hist)
    ones = jnp.ones((16,), jnp.float32)

    def eloop(j, carry):
        chunk = j * 32 + w

        @pl.when(chunk < nchunks)
        def _():
            pltpu.sync_copy(colr_hbm.at[chunk], colbuf)
            for v in range(_HC // 16):
                cv = colbuf[pl.ds(v * 16, 16)]
                plsc.addupdate_scatter(hist, [cv], ones)

        return carry

    lax.fori_loop(0, jmax, eloop, 0)
    pltpu.sync_copy(hist, out_hbm.at[w])


def _hist_stage(col, n):
    e = col.shape[0]
    hc = _HC if e % _HC == 0 else _EC
    colr = col.reshape(e // hc, hc)
    hr = (n + 15) // 16 * 16
    zeros = jnp.zeros((hr,), jnp.float32)
    mesh = plsc.VectorSubcoreMesh(core_axis_name="c", subcore_axis_name="s")
    hists = pl.kernel(
        _hist_body,
        out_type=jax.ShapeDtypeStruct((32, hr), jnp.float32),
        mesh=mesh,
        compiler_params=pltpu.CompilerParams(needs_layout_passes=False),
        scratch_types=[
            pltpu.VMEM((hr,), jnp.float32),
            pltpu.VMEM((hc,), jnp.int32),
        ],
    )(colr, zeros)
    return hists


def _node_body(x_ref, agg_ref, hist_ref, b_ref, n2w1t, n2w1b, n2b1, n2w2, n2b2,
               gw1, gb1, gw2p, gb2p, u_ref, gsum, gcnt, *, nblocks, num_graphs):
    i = pl.program_id(0)

    @pl.when(i == 0)
    def _():
        gsum[...] = jnp.zeros_like(gsum)
        gcnt[...] = jnp.zeros_like(gcnt)

    x = x_ref[...]
    aggs = agg_ref[...]
    cnt = jnp.sum(hist_ref[0], axis=0)
    agg = aggs / jnp.maximum(cnt, 1.0)[:, None]
    h = jnp.maximum(_dot(x, n2w1t[...]) + _dot(agg, n2w1b[...]) + n2b1[...], 0.0)
    x2 = _dot(h, n2w2[...]) + n2b2[...]
    b = b_ref[0, 0, :]
    bn = x.shape[0]
    onehot = (b[:, None] == jax.lax.broadcasted_iota(jnp.int32, (bn, num_graphs), 1)
              ).astype(jnp.float32)
    seg = lambda v: jax.lax.dot_general(onehot, v, (((0,), (0,)), ((), ())),
                                        preferred_element_type=jnp.float32)
    gsum[...] += seg(x2)
    gcnt[...] += seg(jnp.ones_like(x2))

    @pl.when(i == nblocks - 1)
    def _():
        gmean = gsum[...] / jnp.maximum(gcnt[...], 1.0)
        hg = jnp.maximum(_dot(gmean, gw1[...]) + gb1[...], 0.0)
        u_ref[...] = _dot(hg, gw2p[...]) + gb2p[...]


def _node_stage(x, agg, hists, batch, n2w1, n2b1, n2w2, n2b2, gw1, gb1, gw2,
                gb2, block_n, num_graphs):
    n = x.shape[0]
    grid = n // block_n
    batch3d = batch.reshape(grid, 1, block_n)
    hist3d = hists[:, :n].reshape(32, grid, block_n).transpose(1, 0, 2)
    gw2p = jnp.zeros((128, 128), jnp.float32).at[:, :2].set(gw2)
    gb2p = jnp.zeros((1, 128), jnp.float32).at[0, :2].set(gb2)
    full = lambda s: pl.BlockSpec(s, lambda i: (0,) * len(s))
    u_full = pl.pallas_call(
        functools.partial(_node_body, nblocks=grid, num_graphs=num_graphs),
        grid=(grid,),
        in_specs=[
            pl.BlockSpec((block_n, 48), lambda i: (i, 0)),
            pl.BlockSpec((block_n, 128), lambda i: (i, 0)),
            pl.BlockSpec((1, 32, block_n), lambda i: (i, 0, 0)),
            pl.BlockSpec((1, 1, block_n), lambda i: (i, 0, 0)),
            full((48, 128)), full((128, 128)), full((1, 128)),
            full((128, 128)), full((1, 128)),
            full((128, 128)), full((1, 128)), full((128, 128)), full((1, 128)),
        ],
        out_specs=pl.BlockSpec((num_graphs, 128), lambda i: (0, 0)),
        out_shape=jax.ShapeDtypeStruct((num_graphs, 128), jnp.float32),
        scratch_shapes=[
            pltpu.VMEM((num_graphs, 128), jnp.float32),
            pltpu.VMEM((num_graphs, 128), jnp.float32),
        ],
    )(x, agg, hist3d, batch3d, n2w1[:48], n2w1[48:], n2b1.reshape(1, 128),
      n2w2, n2b2.reshape(1, 128), gw1, gb1.reshape(1, 128), gw2p, gb2p)
    return u_full[:, :2]


def kernel(x, edge_index, batch, ew1, eb1, ew2, eb2, n1w1, n1b1, n1w2, n1b2,
           n2w1, n2b1, n2w2, n2b2, gw1, gb1, gw2, gb2):
    n = x.shape[0]
    e = edge_index.shape[1]
    row, col = edge_index[0], edge_index[1]

    if e % _GB == 0:
        xr, xc = _gather_stage(x, row, col)
    else:
        xp = jnp.zeros((n, 128), jnp.float32).at[:, :48].set(x)
        xr = jnp.take(xp, row, axis=0)
        xc = jnp.take(xp, col, axis=0)

    block_e = 2000 if e % 2000 == 0 else e
    ea, m = _edge_stage(xr, xc, ew1, eb1, ew2, eb2, n1w1, n1b1, n1w2, n1b2,
                        block_e)

    agg = _scatter_stage(m, col, n)
    hists = _hist_stage(col, n)

    block_n = 2000 if n % 2000 == 0 else n
    u = _node_stage(x, agg, hists, batch, n2w1, n2b1, n2w2, n2b2, gw1, gb1,
                    gw2, gb2, block_n, 64)
    return (u, ea)


# speedup vs baseline: 3.4857x; 1.3016x over previous
import functools

import jax
import jax.numpy as jnp
from jax import lax
from jax.experimental import pallas as pl
from jax.experimental.pallas import tpu as pltpu
from jax.experimental.pallas import tpu_sc as plsc


def _dot(a, b):
    return jax.lax.dot_general(a, b, (((1,), (0,)), ((), ())),
                               preferred_element_type=jnp.float32)


def _edge_body(xr_ref, xc_ref, ew1t, ew1b, eb1, ew2, eb2,
               n1w1t, n1w1b, n1b1, n1w2, n1b2, ea_ref, m_ref):
    bf = jnp.bfloat16
    xr = xr_ref[...]
    xc = xc_ref[...]
    h1 = jnp.maximum(_dot(xr, ew1t[...]) + _dot(xc, ew1b[...]) + eb1[...],
                     0.0)
    ea = _dot(h1, ew2[...]) + eb2[...]
    ea_ref[...] = ea
    h2 = jnp.maximum(_dot(xr.astype(bf), n1w1t[...].astype(bf))
                     + _dot(ea.astype(bf), n1w1b[...].astype(bf)) + n1b1[...], 0.0)
    m_ref[...] = _dot(h2.astype(bf), n1w2[...].astype(bf)) + n1b2[...]


def _edge_stage(xr, xc, ew1, eb1, ew2, eb2, n1w1, n1b1, n1w2, n1b2, block_e):
    e = xr.shape[0]
    grid = e // block_e
    full = lambda s: pl.BlockSpec(s, lambda i: (0, 0))
    ea, m = pl.pallas_call(
        _edge_body,
        grid=(grid,),
        in_specs=[
            pl.BlockSpec((block_e, 128), lambda i: (i, 0)),
            pl.BlockSpec((block_e, 128), lambda i: (i, 0)),
            full((128, 128)), full((128, 128)), full((1, 128)),
            full((128, 128)), full((1, 128)),
            full((128, 128)), full((128, 128)), full((1, 128)),
            full((128, 128)), full((1, 128)),
        ],
        out_specs=[
            pl.BlockSpec((block_e, 128), lambda i: (i, 0)),
            pl.BlockSpec((block_e, 128), lambda i: (i, 0)),
        ],
        out_shape=[
            jax.ShapeDtypeStruct((e, 128), jnp.float32),
            jax.ShapeDtypeStruct((e, 128), jnp.float32),
        ],
    )(xr, xc,
      jnp.zeros((128, 128), jnp.float32).at[:48].set(ew1[:48]),
      jnp.zeros((128, 128), jnp.float32).at[:48].set(ew1[48:]),
      eb1.reshape(1, 128), ew2, eb2.reshape(1, 128),
      jnp.zeros((128, 128), jnp.float32).at[:48].set(n1w1[:48]),
      n1w1[48:], n1b1.reshape(1, 128), n1w2, n1b2.reshape(1, 128))
    return ea, m


_GB = 128


def _gather_body(x_hbm, rowr_hbm, colr_hbm, xr_hbm, xc_hbm,
                 idxr, idxc, gr, gc, si0, si1, si2, si3, sg0, sg1, sw0, sw1):
    c = lax.axis_index("c")
    s = lax.axis_index("s")
    w = s * 2 + c
    nsc = rowr_hbm.shape[0]
    nbt = (nsc + 31) // 32
    sis = [si0, si1, si2, si3]
    sgs = [sg0, sg1]
    sws = [sw0, sw1]

    def valid(b):
        return (b >= 0) & (b < nbt) & (b * 32 + w < nsc)

    def fire_idx(b, v):
        @pl.when(valid(b))
        def _():
            chunk = b * 32 + w
            pltpu.async_copy(rowr_hbm.at[chunk], idxr.at[v], sis[v])
            pltpu.async_copy(colr_hbm.at[chunk], idxc.at[v], sis[v])

    def wait_idx(b, v):
        @pl.when(valid(b))
        def _():
            pltpu.make_async_copy(rowr_hbm.at[0], idxr.at[v], sis[v]).wait()
            pltpu.make_async_copy(colr_hbm.at[0], idxc.at[v], sis[v]).wait()

    def fire_gathers(b, u, v):
        @pl.when(valid(b))
        def _():
            pltpu.async_copy(x_hbm.at[idxr.at[v]],
                             gr.at[pl.ds(u * _GB, _GB)], sgs[u])
            pltpu.async_copy(x_hbm.at[idxc.at[v]],
                             gc.at[pl.ds(u * _GB, _GB)], sgs[u])

    def wait_gathers(b, u):
        @pl.when(valid(b))
        def _():
            pltpu.make_async_copy(x_hbm.at[idxr.at[0]],
                                  gr.at[pl.ds(u * _GB, _GB)], sgs[u]).wait()
            pltpu.make_async_copy(x_hbm.at[idxc.at[0]],
                                  gc.at[pl.ds(u * _GB, _GB)], sgs[u]).wait()

    def fire_write(b, u):
        @pl.when(valid(b))
        def _():
            chunk = b * 32 + w
            pltpu.async_copy(gr.at[pl.ds(u * _GB, _GB)],
                             xr_hbm.at[pl.ds(chunk * _GB, _GB)], sws[u])
            pltpu.async_copy(gc.at[pl.ds(u * _GB, _GB)],
                             xc_hbm.at[pl.ds(chunk * _GB, _GB)], sws[u])

    def wait_write(b, u):
        @pl.when(valid(b))
        def _():
            pltpu.make_async_copy(gr.at[pl.ds(u * _GB, _GB)],
                                  xr_hbm.at[pl.ds(0, _GB)], sws[u]).wait()
            pltpu.make_async_copy(gc.at[pl.ds(u * _GB, _GB)],
                                  xc_hbm.at[pl.ds(0, _GB)], sws[u]).wait()

    def step(b):
        if b >= 1:
            wait_gathers(b - 1, (b - 1) % 2)
            fire_write(b - 1, (b - 1) % 2)
        if b >= 2:
            wait_write(b - 2, b % 2)
        wait_idx(b, b % 4)
        fire_gathers(b, b % 2, b % 4)
        fire_idx(b + 2, (b + 2) % 4)

    fire_idx(0, 0)
    fire_idx(1, 1)
    for b in range(4):
        step(b)

    def body(t, carry):
        for u in range(4):
            b = t * 4 + u
            wait_gathers(b - 1, (u + 1) % 2)
            fire_write(b - 1, (u + 1) % 2)
            wait_write(b - 2, u % 2)
            wait_idx(b, u)
            fire_gathers(b, u % 2, u)
            fire_idx(b + 2, (u + 2) % 4)
        return carry

    lax.fori_loop(1, (nbt + 3) // 4, body, 0)
    last = ((nbt + 3) // 4) * 4 - 1
    wait_gathers(last, last % 2)
    fire_write(last, last % 2)
    wait_write(last - 1, (last - 1) % 2)
    wait_write(last, last % 2)


def _gather_stage(x, row, col):
    e = row.shape[0]
    n = x.shape[0]
    xp = jnp.zeros((n, 128), jnp.float32).at[:, :x.shape[1]].set(x)
    rowr = row.reshape(e // _GB, _GB)
    colr = col.reshape(e // _GB, _GB)
    mesh = plsc.VectorSubcoreMesh(core_axis_name="c", subcore_axis_name="s")
    xr, xc = pl.kernel(
        _gather_body,
        out_type=[jax.ShapeDtypeStruct((e, 128), jnp.float32),
                  jax.ShapeDtypeStruct((e, 128), jnp.float32)],
        mesh=mesh,
        scratch_types=[
            pltpu.VMEM((4, _GB), jnp.int32),
            pltpu.VMEM((4, _GB), jnp.int32),
            pltpu.VMEM((2 * _GB, 128), jnp.float32),
            pltpu.VMEM((2 * _GB, 128), jnp.float32),
        ] + [pltpu.SemaphoreType.DMA] * 8,
    )(xp, rowr, colr)
    return xr, xc


_EC = 64


_SLOTS = 3


def _scatter_body(m_hbm, colr_hbm, zeros_hbm, agg_hbm, acc, colbuf, libuf, mbuf,
                  sg0, sg1, sg2, sa0, sa1, sa2, *, nch, stripe):
    c = lax.axis_index("c")
    s = lax.axis_index("s")
    nchunks = colr_hbm.shape[0]
    nbt = (nchunks + 15) // 16
    full_tiles = nch // stripe
    partial = nch - full_tiles * stripe
    sgs = [sg0, sg1, sg2]
    sas = [sa0, sa1, sa2]

    def valid(b):
        return (b >= 0) & (b < nbt) & (s * nbt + b < nchunks)

    def fire_stage(b, u):
        chunk = s * nbt + b

        @pl.when(valid(b))
        def _():
            pltpu.async_copy(colr_hbm.at[chunk], colbuf.at[u], sgs[u])
            pltpu.async_copy(m_hbm.at[pl.ds(chunk * _EC, _EC)],
                             mbuf.at[pl.ds(u * _EC, _EC)], sgs[u])

    def wait_stage(b, u):
        @pl.when(valid(b))
        def _():
            pltpu.make_async_copy(colr_hbm.at[0], colbuf.at[u], sgs[u]).wait()
            pltpu.make_async_copy(m_hbm.at[pl.ds(0, _EC)],
                                  mbuf.at[pl.ds(u * _EC, _EC)], sgs[u]).wait()

    def fire_add(b, u, base):
        @pl.when(valid(b))
        def _():
            for v in range(_EC // 16):
                cv = colbuf[u, pl.ds(v * 16, 16)]
                li = cv - base
                memb = (li >= 0) & (li < nch)
                libuf[u, pl.ds(v * 16, 16)] = jnp.where(
                    memb, li, nch + (cv & 31))
            pltpu.async_copy(mbuf.at[pl.ds(u * _EC, _EC)], acc.at[libuf.at[u]],
                             sas[u], add=True)

    def wait_add(b, u):
        @pl.when(valid(b))
        def _():
            pltpu.make_async_copy(m_hbm.at[pl.ds(0, _EC)],
                                  mbuf.at[pl.ds(u * _EC, _EC)], sas[u]).wait()

    nbody = (nbt + _SLOTS - 1) // _SLOTS
    for p in range(2):
        base = (2 * p + c) * nch
        pltpu.sync_copy(zeros_hbm.at[pl.ds(s * stripe, stripe)],
                        acc.at[pl.ds(s * stripe, stripe)])
        plsc.subcore_barrier()

        fire_stage(0, 0)
        fire_stage(1, 1)
        for b in range(_SLOTS):
            if b >= 1:
                wait_add(b - 1, (b - 1) % _SLOTS)
            fire_stage(b + 2, (b + 2) % _SLOTS)
            wait_stage(b, b % _SLOTS)
            fire_add(b, b % _SLOTS, base)

        def body(t, carry):
            for u in range(_SLOTS):
                b = t * _SLOTS + u
                wait_add(b - 1, (u + 2) % _SLOTS)
                fire_stage(b + 2, (u + 2) % _SLOTS)
                wait_stage(b, u)
                fire_add(b, u, base)
            return carry

        lax.fori_loop(1, nbody, body, 0)
        for b in range(_SLOTS * nbody - 1, nbt):
            wait_add(b, b % _SLOTS)
        plsc.subcore_barrier()
        start = s * stripe

        @pl.when(s < full_tiles)
        def _():
            pltpu.sync_copy(acc.at[pl.ds(start, stripe)],
                            agg_hbm.at[pl.ds(base + start, stripe)])

        if partial:
            @pl.when(s == full_tiles)
            def _():
                pltpu.sync_copy(acc.at[pl.ds(start, partial)],
                                agg_hbm.at[pl.ds(base + start, partial)])

        plsc.subcore_barrier()


def _scatter_stage(m, col, n):
    e = m.shape[0]
    colr = col.reshape(e // _EC, _EC)
    nch = (-(-n // 4) + 7) // 8 * 8
    stripe = (-(-(nch + 32) // 16) + 7) // 8 * 8
    accrows = stripe * 16
    zeros = jnp.zeros((accrows, 128), jnp.float32)
    mesh = plsc.VectorSubcoreMesh(core_axis_name="c", subcore_axis_name="s")
    agg = pl.kernel(
        functools.partial(_scatter_body, nch=nch, stripe=stripe),
        out_type=jax.ShapeDtypeStruct((4 * nch, 128), jnp.float32),
        mesh=mesh,
        scratch_types=[
            pltpu.VMEM_SHARED((accrows, 128), jnp.float32),
            pltpu.VMEM((_SLOTS, _EC), jnp.int32),
            pltpu.VMEM((_SLOTS, _EC), jnp.int32),
            pltpu.VMEM((_SLOTS * _EC, 128), jnp.float32),
        ] + [pltpu.SemaphoreType.DMA] * 6,
    )(m, colr, zeros)
    return agg


_HC = 1600


def _hist_body(colr_hbm, zeros_hbm, out_hbm, hist, colbuf):
    c = lax.axis_index("c")
    s = lax.axis_index("s")
    w = s * 2 + c
    nchunks = colr_hbm.shape[0]
    jmax = (nchunks + 31) // 32
    pltpu.sync_copy(zeros_hbm, hist)
    ones = jnp.ones((16,), jnp.float32)

    def eloop(j, carry):
        chunk = j * 32 + w

        @pl.when(chunk < nchunks)
        def _():
            pltpu.sync_copy(colr_hbm.at[chunk], colbuf)
            for v in range(_HC // 16):
                cv = colbuf[pl.ds(v * 16, 16)]
                plsc.addupdate_scatter(hist, [cv], ones)

        return carry

    lax.fori_loop(0, jmax, eloop, 0)
    pltpu.sync_copy(hist, out_hbm.at[w])


def _hist_stage(col, n):
    e = col.shape[0]
    hc = _HC if e % _HC == 0 else _EC
    colr = col.reshape(e // hc, hc)
    hr = (n + 15) // 16 * 16
    zeros = jnp.zeros((hr,), jnp.float32)
    mesh = plsc.VectorSubcoreMesh(core_axis_name="c", subcore_axis_name="s")
    hists = pl.kernel(
        _hist_body,
        out_type=jax.ShapeDtypeStruct((32, hr), jnp.float32),
        mesh=mesh,
        compiler_params=pltpu.CompilerParams(needs_layout_passes=False),
        scratch_types=[
            pltpu.VMEM((hr,), jnp.float32),
            pltpu.VMEM((hc,), jnp.int32),
        ],
    )(colr, zeros)
    return hists


def _node_body(x_ref, agg_ref, hist_ref, b_ref, n2w1t, n2w1b, n2b1, n2w2, n2b2,
               gw1, gb1, gw2p, gb2p, u_ref, gsum, gcnt, *, nblocks, num_graphs):
    i = pl.program_id(0)

    @pl.when(i == 0)
    def _():
        gsum[...] = jnp.zeros_like(gsum)
        gcnt[...] = jnp.zeros_like(gcnt)

    x = x_ref[...]
    aggs = agg_ref[...]
    cnt = jnp.sum(hist_ref[0], axis=0)
    agg = aggs / jnp.maximum(cnt, 1.0)[:, None]
    h = jnp.maximum(_dot(x, n2w1t[...]) + _dot(agg, n2w1b[...]) + n2b1[...], 0.0)
    x2 = _dot(h, n2w2[...]) + n2b2[...]
    b = b_ref[0, 0, :]
    bn = x.shape[0]
    onehot = (b[:, None] == jax.lax.broadcasted_iota(jnp.int32, (bn, num_graphs), 1)
              ).astype(jnp.float32)
    seg = lambda v: jax.lax.dot_general(onehot, v, (((0,), (0,)), ((), ())),
                                        preferred_element_type=jnp.float32)
    gsum[...] += seg(x2)
    gcnt[...] += seg(jnp.ones_like(x2))

    @pl.when(i == nblocks - 1)
    def _():
        gmean = gsum[...] / jnp.maximum(gcnt[...], 1.0)
        hg = jnp.maximum(_dot(gmean, gw1[...]) + gb1[...], 0.0)
        u_ref[...] = _dot(hg, gw2p[...]) + gb2p[...]


def _node_stage(x, agg, hists, batch, n2w1, n2b1, n2w2, n2b2, gw1, gb1, gw2,
                gb2, block_n, num_graphs):
    n = x.shape[0]
    grid = n // block_n
    batch3d = batch.reshape(grid, 1, block_n)
    hist3d = hists[:, :n].reshape(32, grid, block_n).transpose(1, 0, 2)
    gw2p = jnp.zeros((128, 128), jnp.float32).at[:, :2].set(gw2)
    gb2p = jnp.zeros((1, 128), jnp.float32).at[0, :2].set(gb2)
    full = lambda s: pl.BlockSpec(s, lambda i: (0,) * len(s))
    u_full = pl.pallas_call(
        functools.partial(_node_body, nblocks=grid, num_graphs=num_graphs),
        grid=(grid,),
        in_specs=[
            pl.BlockSpec((block_n, 48), lambda i: (i, 0)),
            pl.BlockSpec((block_n, 128), lambda i: (i, 0)),
            pl.BlockSpec((1, 32, block_n), lambda i: (i, 0, 0)),
            pl.BlockSpec((1, 1, block_n), lambda i: (i, 0, 0)),
            full((48, 128)), full((128, 128)), full((1, 128)),
            full((128, 128)), full((1, 128)),
            full((128, 128)), full((1, 128)), full((128, 128)), full((1, 128)),
        ],
        out_specs=pl.BlockSpec((num_graphs, 128), lambda i: (0, 0)),
        out_shape=jax.ShapeDtypeStruct((num_graphs, 128), jnp.float32),
        scratch_shapes=[
            pltpu.VMEM((num_graphs, 128), jnp.float32),
            pltpu.VMEM((num_graphs, 128), jnp.float32),
        ],
    )(x, agg, hist3d, batch3d, n2w1[:48], n2w1[48:], n2b1.reshape(1, 128),
      n2w2, n2b2.reshape(1, 128), gw1, gb1.reshape(1, 128), gw2p, gb2p)
    return u_full[:, :2]


def kernel(x, edge_index, batch, ew1, eb1, ew2, eb2, n1w1, n1b1, n1w2, n1b2,
           n2w1, n2b1, n2w2, n2b2, gw1, gb1, gw2, gb2):
    n = x.shape[0]
    e = edge_index.shape[1]
    row, col = edge_index[0], edge_index[1]

    if e % _GB == 0:
        xr, xc = _gather_stage(x, row, col)
    else:
        xp = jnp.zeros((n, 128), jnp.float32).at[:, :48].set(x)
        xr = jnp.take(xp, row, axis=0)
        xc = jnp.take(xp, col, axis=0)

    block_e = 2000 if e % 2000 == 0 else e
    ea, m = _edge_stage(xr, xc, ew1, eb1, ew2, eb2, n1w1, n1b1, n1w2, n1b2,
                        block_e)

    agg = _scatter_stage(m, col, n)
    hists = _hist_stage(col, n)

    block_n = 2000 if n % 2000 == 0 else n
    u = _node_stage(x, agg, hists, batch, n2w1, n2b1, n2w2, n2b2, gw1, gb1,
                    gw2, gb2, block_n, 64)
    return (u, ea)


# speedup vs baseline: 3.6090x; 1.0354x over previous
import functools

import jax
import jax.numpy as jnp
from jax import lax
from jax.experimental import pallas as pl
from jax.experimental.pallas import tpu as pltpu
from jax.experimental.pallas import tpu_sc as plsc


def _dot(a, b):
    return jax.lax.dot_general(a, b, (((1,), (0,)), ((), ())),
                               preferred_element_type=jnp.float32)


def _edge_body(xr_ref, xc_ref, ew1t, ew1b, eb1, ew2, eb2,
               n1w1t, n1w1b, n1b1, n1w2, n1b2, ea_ref, m_ref):
    xr = xr_ref[...]
    xc = xc_ref[...]
    h1 = jnp.maximum(_dot(xr, ew1t[...]) + _dot(xc, ew1b[...]) + eb1[...],
                     0.0)
    ea = _dot(h1, ew2[...]) + eb2[...]
    ea_ref[...] = ea
    h2 = jnp.maximum(_dot(xr, n1w1t[...]) + _dot(ea, n1w1b[...]) + n1b1[...],
                     0.0)
    m_ref[...] = _dot(h2, n1w2[...]) + n1b2[...]


def _edge_stage(xr, xc, ew1, eb1, ew2, eb2, n1w1, n1b1, n1w2, n1b2, block_e):
    e = xr.shape[0]
    grid = e // block_e
    full = lambda s: pl.BlockSpec(s, lambda i: (0, 0))
    ea, m = pl.pallas_call(
        _edge_body,
        grid=(grid,),
        in_specs=[
            pl.BlockSpec((block_e, 128), lambda i: (i, 0)),
            pl.BlockSpec((block_e, 128), lambda i: (i, 0)),
            full((128, 128)), full((128, 128)), full((1, 128)),
            full((128, 128)), full((1, 128)),
            full((128, 128)), full((128, 128)), full((1, 128)),
            full((128, 128)), full((1, 128)),
        ],
        out_specs=[
            pl.BlockSpec((block_e, 128), lambda i: (i, 0)),
            pl.BlockSpec((block_e, 128), lambda i: (i, 0)),
        ],
        out_shape=[
            jax.ShapeDtypeStruct((e, 128), jnp.float32),
            jax.ShapeDtypeStruct((e, 128), jnp.float32),
        ],
    )(xr, xc,
      jnp.zeros((128, 128), jnp.float32).at[:48].set(ew1[:48]),
      jnp.zeros((128, 128), jnp.float32).at[:48].set(ew1[48:]),
      eb1.reshape(1, 128), ew2, eb2.reshape(1, 128),
      jnp.zeros((128, 128), jnp.float32).at[:48].set(n1w1[:48]),
      n1w1[48:], n1b1.reshape(1, 128), n1w2, n1b2.reshape(1, 128))
    return ea, m


_GB = 128


def _gather_body(x_hbm, rowr_hbm, colr_hbm, xr_hbm, xc_hbm,
                 idxr, idxc, gr, gc, si0, si1, si2, si3, sg0, sg1, sw0, sw1):
    c = lax.axis_index("c")
    s = lax.axis_index("s")
    w = s * 2 + c
    nsc = rowr_hbm.shape[0]
    nbt = (nsc + 31) // 32
    sis = [si0, si1, si2, si3]
    sgs = [sg0, sg1]
    sws = [sw0, sw1]

    def valid(b):
        return (b >= 0) & (b < nbt) & (b * 32 + w < nsc)

    def fire_idx(b, v):
        @pl.when(valid(b))
        def _():
            chunk = b * 32 + w
            pltpu.async_copy(rowr_hbm.at[chunk], idxr.at[v], sis[v])
            pltpu.async_copy(colr_hbm.at[chunk], idxc.at[v], sis[v])

    def wait_idx(b, v):
        @pl.when(valid(b))
        def _():
            pltpu.make_async_copy(rowr_hbm.at[0], idxr.at[v], sis[v]).wait()
            pltpu.make_async_copy(colr_hbm.at[0], idxc.at[v], sis[v]).wait()

    def fire_gathers(b, u, v):
        @pl.when(valid(b))
        def _():
            pltpu.async_copy(x_hbm.at[idxr.at[v]],
                             gr.at[pl.ds(u * _GB, _GB)], sgs[u])
            pltpu.async_copy(x_hbm.at[idxc.at[v]],
                             gc.at[pl.ds(u * _GB, _GB)], sgs[u])

    def wait_gathers(b, u):
        @pl.when(valid(b))
        def _():
            pltpu.make_async_copy(x_hbm.at[idxr.at[0]],
                                  gr.at[pl.ds(u * _GB, _GB)], sgs[u]).wait()
            pltpu.make_async_copy(x_hbm.at[idxc.at[0]],
                                  gc.at[pl.ds(u * _GB, _GB)], sgs[u]).wait()

    def fire_write(b, u):
        @pl.when(valid(b))
        def _():
            chunk = b * 32 + w
            pltpu.async_copy(gr.at[pl.ds(u * _GB, _GB)],
                             xr_hbm.at[pl.ds(chunk * _GB, _GB)], sws[u])
            pltpu.async_copy(gc.at[pl.ds(u * _GB, _GB)],
                             xc_hbm.at[pl.ds(chunk * _GB, _GB)], sws[u])

    def wait_write(b, u):
        @pl.when(valid(b))
        def _():
            pltpu.make_async_copy(gr.at[pl.ds(u * _GB, _GB)],
                                  xr_hbm.at[pl.ds(0, _GB)], sws[u]).wait()
            pltpu.make_async_copy(gc.at[pl.ds(u * _GB, _GB)],
                                  xc_hbm.at[pl.ds(0, _GB)], sws[u]).wait()

    def step(b):
        if b >= 1:
            wait_gathers(b - 1, (b - 1) % 2)
            fire_write(b - 1, (b - 1) % 2)
        if b >= 2:
            wait_write(b - 2, b % 2)
        wait_idx(b, b % 4)
        fire_gathers(b, b % 2, b % 4)
        fire_idx(b + 2, (b + 2) % 4)

    fire_idx(0, 0)
    fire_idx(1, 1)
    for b in range(4):
        step(b)

    def body(t, carry):
        for u in range(4):
            b = t * 4 + u
            wait_gathers(b - 1, (u + 1) % 2)
            fire_write(b - 1, (u + 1) % 2)
            wait_write(b - 2, u % 2)
            wait_idx(b, u)
            fire_gathers(b, u % 2, u)
            fire_idx(b + 2, (u + 2) % 4)
        return carry

    lax.fori_loop(1, (nbt + 3) // 4, body, 0)
    last = ((nbt + 3) // 4) * 4 - 1
    wait_gathers(last, last % 2)
    fire_write(last, last % 2)
    wait_write(last - 1, (last - 1) % 2)
    wait_write(last, last % 2)


def _gather_stage(x, row, col):
    e = row.shape[0]
    n = x.shape[0]
    xp = jnp.zeros((n, 128), jnp.float32).at[:, :x.shape[1]].set(x)
    rowr = row.reshape(e // _GB, _GB)
    colr = col.reshape(e // _GB, _GB)
    mesh = plsc.VectorSubcoreMesh(core_axis_name="c", subcore_axis_name="s")
    xr, xc = pl.kernel(
        _gather_body,
        out_type=[jax.ShapeDtypeStruct((e, 128), jnp.float32),
                  jax.ShapeDtypeStruct((e, 128), jnp.float32)],
        mesh=mesh,
        scratch_types=[
            pltpu.VMEM((4, _GB), jnp.int32),
            pltpu.VMEM((4, _GB), jnp.int32),
            pltpu.VMEM((2 * _GB, 128), jnp.float32),
            pltpu.VMEM((2 * _GB, 128), jnp.float32),
        ] + [pltpu.SemaphoreType.DMA] * 8,
    )(xp, rowr, colr)
    return xr, xc


_EC = 64


_SLOTS = 3


def _scatter_body(m_hbm, colr_hbm, zeros_hbm, agg_hbm, acc, colbuf, libuf, mbuf,
                  sg0, sg1, sg2, sa0, sa1, sa2, *, nch, stripe):
    c = lax.axis_index("c")
    s = lax.axis_index("s")
    nchunks = colr_hbm.shape[0]
    nbt = (nchunks + 15) // 16
    full_tiles = nch // stripe
    partial = nch - full_tiles * stripe
    sgs = [sg0, sg1, sg2]
    sas = [sa0, sa1, sa2]

    def valid(b):
        return (b >= 0) & (b < nbt) & (s * nbt + b < nchunks)

    def fire_stage(b, u):
        chunk = s * nbt + b

        @pl.when(valid(b))
        def _():
            pltpu.async_copy(colr_hbm.at[chunk], colbuf.at[u], sgs[u])
            pltpu.async_copy(m_hbm.at[pl.ds(chunk * _EC, _EC)],
                             mbuf.at[pl.ds(u * _EC, _EC)], sgs[u])

    def wait_stage(b, u):
        @pl.when(valid(b))
        def _():
            pltpu.make_async_copy(colr_hbm.at[0], colbuf.at[u], sgs[u]).wait()
            pltpu.make_async_copy(m_hbm.at[pl.ds(0, _EC)],
                                  mbuf.at[pl.ds(u * _EC, _EC)], sgs[u]).wait()

    def fire_add(b, u, base):
        @pl.when(valid(b))
        def _():
            for v in range(_EC // 16):
                cv = colbuf[u, pl.ds(v * 16, 16)]
                li = cv - base
                memb = (li >= 0) & (li < nch)
                libuf[u, pl.ds(v * 16, 16)] = jnp.where(
                    memb, li, nch + (cv & 31))
            pltpu.async_copy(mbuf.at[pl.ds(u * _EC, _EC)], acc.at[libuf.at[u]],
                             sas[u], add=True)

    def wait_add(b, u):
        @pl.when(valid(b))
        def _():
            pltpu.make_async_copy(m_hbm.at[pl.ds(0, _EC)],
                                  mbuf.at[pl.ds(u * _EC, _EC)], sas[u]).wait()

    nbody = (nbt + _SLOTS - 1) // _SLOTS
    for p in range(2):
        base = (2 * p + c) * nch
        pltpu.sync_copy(zeros_hbm.at[pl.ds(s * stripe, stripe)],
                        acc.at[pl.ds(s * stripe, stripe)])
        plsc.subcore_barrier()

        fire_stage(0, 0)
        fire_stage(1, 1)
        for b in range(_SLOTS):
            if b >= 1:
                wait_add(b - 1, (b - 1) % _SLOTS)
            fire_stage(b + 2, (b + 2) % _SLOTS)
            wait_stage(b, b % _SLOTS)
            fire_add(b, b % _SLOTS, base)

        def body(t, carry):
            for u in range(_SLOTS):
                b = t * _SLOTS + u
                wait_add(b - 1, (u + 2) % _SLOTS)
                fire_stage(b + 2, (u + 2) % _SLOTS)
                wait_stage(b, u)
                fire_add(b, u, base)
            return carry

        lax.fori_loop(1, nbody, body, 0)
        for b in range(_SLOTS * nbody - 1, nbt):
            wait_add(b, b % _SLOTS)
        plsc.subcore_barrier()
        start = s * stripe

        @pl.when(s < full_tiles)
        def _():
            pltpu.sync_copy(acc.at[pl.ds(start, stripe)],
                            agg_hbm.at[pl.ds(base + start, stripe)])

        if partial:
            @pl.when(s == full_tiles)
            def _():
                pltpu.sync_copy(acc.at[pl.ds(start, partial)],
                                agg_hbm.at[pl.ds(base + start, partial)])

        plsc.subcore_barrier()


def _scatter_stage(m, col, n):
    e = m.shape[0]
    colr = col.reshape(e // _EC, _EC)
    nch = (-(-n // 4) + 7) // 8 * 8
    stripe = (-(-(nch + 32) // 16) + 7) // 8 * 8
    accrows = stripe * 16
    zeros = jnp.zeros((accrows, 128), jnp.float32)
    mesh = plsc.VectorSubcoreMesh(core_axis_name="c", subcore_axis_name="s")
    agg = pl.kernel(
        functools.partial(_scatter_body, nch=nch, stripe=stripe),
        out_type=jax.ShapeDtypeStruct((4 * nch, 128), jnp.float32),
        mesh=mesh,
        scratch_types=[
            pltpu.VMEM_SHARED((accrows, 128), jnp.float32),
            pltpu.VMEM((_SLOTS, _EC), jnp.int32),
            pltpu.VMEM((_SLOTS, _EC), jnp.int32),
            pltpu.VMEM((_SLOTS * _EC, 128), jnp.float32),
        ] + [pltpu.SemaphoreType.DMA] * 6,
    )(m, colr, zeros)
    return agg


_HC = 1600


def _hist_body(colr_hbm, zeros_hbm, out_hbm, hist, colbuf):
    c = lax.axis_index("c")
    s = lax.axis_index("s")
    w = s * 2 + c
    nchunks = colr_hbm.shape[0]
    jmax = (nchunks + 31) // 32
    pltpu.sync_copy(zeros_hbm, hist)
    ones = jnp.ones((16,), jnp.float32)

    def eloop(j, carry):
        chunk = j * 32 + w

        @pl.when(chunk < nchunks)
        def _():
            pltpu.sync_copy(colr_hbm.at[chunk], colbuf)
            for v in range(_HC // 16):
                cv = colbuf[pl.ds(v * 16, 16)]
                plsc.addupdate_scatter(hist, [cv], ones)

        return carry

    lax.fori_loop(0, jmax, eloop, 0)
    pltpu.sync_copy(hist, out_hbm.at[w])


def _hist_stage(col, n):
    e = col.shape[0]
    hc = _HC if e % _HC == 0 else _EC
    colr = col.reshape(e // hc, hc)
    hr = (n + 15) // 16 * 16
    zeros = jnp.zeros((hr,), jnp.float32)
    mesh = plsc.VectorSubcoreMesh(core_axis_name="c", subcore_axis_name="s")
    hists = pl.kernel(
        _hist_body,
        out_type=jax.ShapeDtypeStruct((32, hr), jnp.float32),
        mesh=mesh,
        compiler_params=pltpu.CompilerParams(needs_layout_passes=False),
        scratch_types=[
            pltpu.VMEM((hr,), jnp.float32),
            pltpu.VMEM((hc,), jnp.int32),
        ],
    )(colr, zeros)
    return hists


def _node_body(x_ref, agg_ref, hist_ref, b_ref, n2w1t, n2w1b, n2b1, n2w2, n2b2,
               gw1, gb1, gw2p, gb2p, u_ref, gsum, gcnt, *, nblocks, num_graphs):
    i = pl.program_id(0)

    @pl.when(i == 0)
    def _():
        gsum[...] = jnp.zeros_like(gsum)
        gcnt[...] = jnp.zeros_like(gcnt)

    x = x_ref[...]
    aggs = agg_ref[...]
    cnt = jnp.sum(hist_ref[0], axis=0)
    agg = aggs / jnp.maximum(cnt, 1.0)[:, None]
    h = jnp.maximum(_dot(x, n2w1t[...]) + _dot(agg, n2w1b[...]) + n2b1[...], 0.0)
    x2 = _dot(h, n2w2[...]) + n2b2[...]
    b = b_ref[0, 0, :]
    bn = x.shape[0]
    onehot = (b[:, None] == jax.lax.broadcasted_iota(jnp.int32, (bn, num_graphs), 1)
              ).astype(jnp.float32)
    seg = lambda v: jax.lax.dot_general(onehot, v, (((0,), (0,)), ((), ())),
                                        preferred_element_type=jnp.float32)
    gsum[...] += seg(x2)
    gcnt[...] += seg(jnp.ones_like(x2))

    @pl.when(i == nblocks - 1)
    def _():
        gmean = gsum[...] / jnp.maximum(gcnt[...], 1.0)
        hg = jnp.maximum(_dot(gmean, gw1[...]) + gb1[...], 0.0)
        u_ref[...] = _dot(hg, gw2p[...]) + gb2p[...]


def _node_stage(x, agg, hists, batch, n2w1, n2b1, n2w2, n2b2, gw1, gb1, gw2,
                gb2, block_n, num_graphs):
    n = x.shape[0]
    grid = n // block_n
    batch3d = batch.reshape(grid, 1, block_n)
    hist3d = hists[:, :n].reshape(32, grid, block_n).transpose(1, 0, 2)
    gw2p = jnp.zeros((128, 128), jnp.float32).at[:, :2].set(gw2)
    gb2p = jnp.zeros((1, 128), jnp.float32).at[0, :2].set(gb2)
    full = lambda s: pl.BlockSpec(s, lambda i: (0,) * len(s))
    u_full = pl.pallas_call(
        functools.partial(_node_body, nblocks=grid, num_graphs=num_graphs),
        grid=(grid,),
        in_specs=[
            pl.BlockSpec((block_n, 48), lambda i: (i, 0)),
            pl.BlockSpec((block_n, 128), lambda i: (i, 0)),
            pl.BlockSpec((1, 32, block_n), lambda i: (i, 0, 0)),
            pl.BlockSpec((1, 1, block_n), lambda i: (i, 0, 0)),
            full((48, 128)), full((128, 128)), full((1, 128)),
            full((128, 128)), full((1, 128)),
            full((128, 128)), full((1, 128)), full((128, 128)), full((1, 128)),
        ],
        out_specs=pl.BlockSpec((num_graphs, 128), lambda i: (0, 0)),
        out_shape=jax.ShapeDtypeStruct((num_graphs, 128), jnp.float32),
        scratch_shapes=[
            pltpu.VMEM((num_graphs, 128), jnp.float32),
            pltpu.VMEM((num_graphs, 128), jnp.float32),
        ],
    )(x, agg, hist3d, batch3d, n2w1[:48], n2w1[48:], n2b1.reshape(1, 128),
      n2w2, n2b2.reshape(1, 128), gw1, gb1.reshape(1, 128), gw2p, gb2p)
    return u_full[:, :2]


def kernel(x, edge_index, batch, ew1, eb1, ew2, eb2, n1w1, n1b1, n1w2, n1b2,
           n2w1, n2b1, n2w2, n2b2, gw1, gb1, gw2, gb2):
    n = x.shape[0]
    e = edge_index.shape[1]
    row, col = edge_index[0], edge_index[1]

    if e % _GB == 0:
        xr, xc = _gather_stage(x, row, col)
    else:
        xp = jnp.zeros((n, 128), jnp.float32).at[:, :48].set(x)
        xr = jnp.take(xp, row, axis=0)
        xc = jnp.take(xp, col, axis=0)

    block_e = 2000 if e % 2000 == 0 else e
    ea, m = _edge_stage(xr, xc, ew1, eb1, ew2, eb2, n1w1, n1b1, n1w2, n1b2,
                        block_e)

    agg = _scatter_stage(m, col, n)
    hists = _hist_stage(col, n)

    block_n = 2000 if n % 2000 == 0 else n
    u = _node_stage(x, agg, hists, batch, n2w1, n2b1, n2w2, n2b2, gw1, gb1,
                    gw2, gb2, block_n, 64)
    return (u, ea)
